# baseline (speedup 1.0000x reference)
import functools

import jax
import jax.numpy as jnp
from jax import lax
from jax.experimental import pallas as pl
from jax.experimental.pallas import tpu as pltpu
from jax.experimental.pallas import tpu_sc as plsc

N = 16384
E = 262144
B = 64
NPG = N // B
D = 128
HG = 128
HS = 512
T = 512
K_STEPS = 16
ALPHA = 0.1

NC = 2
NS = 16
L = 16

EPT = E // NS
SELF_PT = N // NS
KCH = 128
QROWS = (EPT + SELF_PT) // KCH
QCAP = QROWS * KCH
DH = 32
NP = 128 // DH

_mesh = plsc.VectorSubcoreMesh(core_axis_name="c", subcore_axis_name="s",
                               num_cores=NC, num_subcores=NS)


def _i32(x):
    return jnp.asarray(x, jnp.int32)


def _popcount16(mask):
    v = jnp.where(mask, 1, 0).astype(jnp.int32)
    pc = v[0]
    for k in range(1, L):
        pc = pc + v[k]
    return pc


def _sc_preprocess(edge_ref, qsa_ref, qd_ref, sbs, sbd):
    cid = lax.axis_index("c")
    sid = lax.axis_index("s")
    iot = lax.iota(jnp.int32, L)

    pltpu.sync_copy(edge_ref.at[0, pl.ds(sid * EPT, EPT)], sbs)
    pltpu.sync_copy(edge_ref.at[1, pl.ds(sid * EPT, EPT)], sbd)

    base = cid * NP * N

    def edge_body(i, _):
        s16 = sbs[pl.ds(i * L, L)]
        sbs[pl.ds(i * L, L)] = s16 + base
        return 0

    lax.fori_loop(0, EPT // L, edge_body, 0)
    pltpu.sync_copy(sbs, qsa_ref.at[cid, sid, pl.ds(0, EPT)])
    pltpu.sync_copy(sbd, qd_ref.at[cid, sid, pl.ds(0, EPT)])

    def self_body(i, _):
        node = sid * SELF_PT + i * L + iot
        sbs[pl.ds(i * L, L)] = node + base
        sbd[pl.ds(i * L, L)] = node
        return 0

    lax.fori_loop(0, SELF_PT // L, self_body, 0)
    pltpu.sync_copy(sbs.at[pl.ds(0, SELF_PT)],
                    qsa_ref.at[cid, sid, pl.ds(EPT, SELF_PT)])
    pltpu.sync_copy(sbd.at[pl.ds(0, SELF_PT)],
                    qd_ref.at[cid, sid, pl.ds(EPT, SELF_PT)])


def _sc_iterate(u_in_ref, qsa_ref, qd_ref, br_ref, cc_ref,
                u_out_ref, qsw_v, qd_v, gbuf, tacc, tcc, tbc, z64, acc_ref,
                gsem, ssem):
    cid = lax.axis_index("c")
    sid = lax.axis_index("s")

    pltpu.sync_copy(qsa_ref.at[cid, sid], qsw_v)
    pltpu.sync_copy(qd_ref.at[cid, sid], qd_v)

    def zb(r, _):
        for j in range(DH // L):
            z64[r, pl.ds(j * L, L)] = jnp.zeros((L,), jnp.float32)
        return 0

    lax.fori_loop(0, 64, zb, 0)

    def az(k, _):
        pltpu.sync_copy(z64, acc_ref.at[pl.ds(sid * 1024 + k * 64, 64)])
        return 0

    lax.fori_loop(0, 16, az, 0)
    plsc.subcore_barrier()

    for p in range(NP):
        if p > 0:
            def shift(r, _):
                for j in range(KCH // L):
                    sl = pl.ds(j * L, L)
                    qsw_v[r, sl] = qsw_v[r, sl] + N
                return 0

            lax.fori_loop(0, QROWS, shift, 0)

        def outer(jj, _):
            base = jj * 8
            gds = []
            for k in range(8):
                gds.append(pltpu.async_copy(
                    u_in_ref.at[qsw_v.at[base + k]], gbuf.at[k], gsem.at[k]))
            sds = []
            for k in range(8):
                gds[k].wait()
                sds.append(pltpu.async_copy(
                    gbuf.at[k], acc_ref.at[qd_v.at[base + k]], ssem.at[k],
                    add=True))
            for k in range(8):
                sds[k].wait()
            return 0

        lax.fori_loop(0, QROWS // 8, outer, 0)
        plsc.subcore_barrier()

        obase = (cid * NP + p) * N

        def blk(b, _):
            lbase = sid * 1024 + b * 64
            pltpu.sync_copy(acc_ref.at[pl.ds(lbase, 64)], tacc)
            pltpu.sync_copy(br_ref.at[pl.ds(lbase, 64)], tbc)
            pltpu.sync_copy(cc_ref.at[pl.ds(obase + lbase, 64)], tcc)
            pltpu.sync_copy(z64, acc_ref.at[pl.ds(lbase, 64)])

            def row(i, _):
                for q in range(DH // L):
                    sl = pl.ds(q * L, L)
                    tacc[i, sl] = tacc[i, sl] * tbc[i, sl] + tcc[i, sl]
                return 0

            lax.fori_loop(0, 64, row, 0)
            pltpu.sync_copy(tacc, u_out_ref.at[pl.ds(obase + lbase, 64)])
            return 0

        lax.fori_loop(0, 16, blk, 0)
        plsc.subcore_barrier()


def _sc_iterate4(u_in_ref, qsa_ref, qd_ref, br_ref, cc_ref,
                 out_ref, qsw_v, qd_v, gbuf, tacc, tcc, tbc, z64,
                 acc_ref, gsem, ssem):
    cid = lax.axis_index("c")
    sid = lax.axis_index("s")
    UTOT = 2 * NP * N

    pltpu.sync_copy(qd_ref.at[cid, sid], qd_v)

    def zb(r, _):
        for j in range(DH // L):
            z64[r, pl.ds(j * L, L)] = jnp.zeros((L,), jnp.float32)
        return 0

    lax.fori_loop(0, 64, zb, 0)

    def az(k, _):
        pltpu.sync_copy(z64, acc_ref.at[pl.ds(sid * 1024 + k * 64, 64)])
        return 0

    lax.fori_loop(0, 16, az, 0)
    plsc.subcore_barrier()

    for it in range(4):
        src_ref = u_in_ref if it == 0 else out_ref
        srcoff = 0 if it == 0 else ((it - 1) & 1) * UTOT
        dstoff = (it & 1) * UTOT

        pltpu.sync_copy(qsa_ref.at[cid, sid], qsw_v)
        if srcoff:
            def shift0(r, _):
                for j in range(KCH // L):
                    sl = pl.ds(j * L, L)
                    qsw_v[r, sl] = qsw_v[r, sl] + srcoff
                return 0

            lax.fori_loop(0, QROWS, shift0, 0)

        for pp in range(NP):
            if pp > 0:
                def shift(r, _):
                    for j in range(KCH // L):
                        sl = pl.ds(j * L, L)
                        qsw_v[r, sl] = qsw_v[r, sl] + N
                    return 0

                lax.fori_loop(0, QROWS, shift, 0)

            def outer(jj, _):
                base = jj * 8
                gds = []
                for k in range(8):
                    gds.append(pltpu.async_copy(
                        src_ref.at[qsw_v.at[base + k]], gbuf.at[k],
                        gsem.at[k]))
                sds = []
                for k in range(8):
                    gds[k].wait()
                    sds.append(pltpu.async_copy(
                        gbuf.at[k], acc_ref.at[qd_v.at[base + k]], ssem.at[k],
                        add=True))
                for k in range(8):
                    sds[k].wait()
                return 0

            lax.fori_loop(0, QROWS // 8, outer, 0)
            plsc.subcore_barrier()

            obase = dstoff + (cid * NP + pp) * N

            def blk(b, _):
                lbase = sid * 1024 + b * 64
                pltpu.sync_copy(acc_ref.at[pl.ds(lbase, 64)], tacc)
                pltpu.sync_copy(br_ref.at[pl.ds(lbase, 64)], tbc)
                pltpu.sync_copy(cc_ref.at[pl.ds((cid * NP + pp) * N + lbase,
                                                64)], tcc)
                pltpu.sync_copy(z64, acc_ref.at[pl.ds(lbase, 64)])

                def row(i, _):
                    for q in range(DH // L):
                        sl = pl.ds(q * L, L)
                        tacc[i, sl] = tacc[i, sl] * tbc[i, sl] + tcc[i, sl]
                    return 0

                lax.fori_loop(0, 64, row, 0)
                pltpu.sync_copy(tacc, out_ref.at[pl.ds(obase + lbase, 64)])
                return 0

            lax.fori_loop(0, 16, blk, 0)
            plsc.subcore_barrier()


_iterate4_call = functools.partial(
    pl.kernel,
    out_type=jax.ShapeDtypeStruct((2 * 2 * NP * N, DH), jnp.float32),
    mesh=_mesh,
    compiler_params=pltpu.CompilerParams(use_tc_tiling_on_sc=False),
    scratch_types=[
        pltpu.VMEM((QROWS, KCH), jnp.int32),
        pltpu.VMEM((QROWS, KCH), jnp.int32),
        pltpu.VMEM((8, KCH, DH), jnp.float32),
        pltpu.VMEM((64, DH), jnp.float32),
        pltpu.VMEM((64, DH), jnp.float32),
        pltpu.VMEM((64, DH), jnp.float32),
        pltpu.VMEM((64, DH), jnp.float32),
        pltpu.VMEM_SHARED((N, DH), jnp.float32),
        pltpu.SemaphoreType.DMA((8,)),
        pltpu.SemaphoreType.DMA((8,)),
    ],
)


_preprocess_call = functools.partial(
    pl.kernel,
    out_type=[
        jax.ShapeDtypeStruct((NC, NS, QCAP), jnp.int32),
        jax.ShapeDtypeStruct((NC, NS, QCAP), jnp.int32),
    ],
    mesh=_mesh,
    scratch_types=[
        pltpu.VMEM((EPT,), jnp.int32),
        pltpu.VMEM((EPT,), jnp.int32),
    ],
)


_iterate_call = functools.partial(
    pl.kernel,
    out_type=jax.ShapeDtypeStruct((2 * NP * N, DH), jnp.float32),
    mesh=_mesh,
    compiler_params=pltpu.CompilerParams(use_tc_tiling_on_sc=False),
    scratch_types=[
        pltpu.VMEM((QROWS, KCH), jnp.int32),
        pltpu.VMEM((QROWS, KCH), jnp.int32),
        pltpu.VMEM((8, KCH, DH), jnp.float32),
        pltpu.VMEM((64, DH), jnp.float32),
        pltpu.VMEM((64, DH), jnp.float32),
        pltpu.VMEM((64, DH), jnp.float32),
        pltpu.VMEM((64, DH), jnp.float32),
        pltpu.VMEM_SHARED((N, DH), jnp.float32),
        pltpu.SemaphoreType.DMA((8,)),
        pltpu.SemaphoreType.DMA((8,)),
    ],
)


def _appnp_sc(h, edge_index):
    qsa, qd = _preprocess_call(_sc_preprocess)(edge_index)
    qsa = qsa.reshape(NC, NS, QROWS, KCH)
    qd = qd.reshape(NC, NS, QROWS, KCH)
    it = _iterate_call(_sc_iterate)
    deg_rows = it(jnp.ones((2 * NP * N, DH), jnp.float32), qsa, qd,
                  jnp.ones((N, DH), jnp.float32),
                  jnp.zeros((2 * NP * N, DH), jnp.float32))
    deg = deg_rows[:N, 0]
    g = deg ** -0.5
    beta_rows = jnp.broadcast_to(((1.0 - ALPHA) / deg)[:, None], (N, DH))
    u = ((h * g[:, None]).reshape(N, 2, NP, DH)
         .transpose(1, 2, 0, 3).reshape(2 * NP * N, DH))
    cc = ALPHA * u
    it4 = _iterate4_call(_sc_iterate4)
    for _ in range(K_STEPS // 4):
        u = it4(u, qsa, qd, beta_rows, cc)[2 * NP * N:]
    hf = (u.reshape(2, NP, N, DH).transpose(2, 0, 1, 3).reshape(N, 2 * HG)
          * jnp.sqrt(deg)[:, None])
    return hf


def _mm_bias_kernel(x_ref, w_ref, b_ref, o_ref, *, act):
    y = jnp.dot(x_ref[...], w_ref[...],
                preferred_element_type=jnp.float32) + b_ref[...]
    if act == "relu":
        y = jnp.maximum(y, 0.0)
    o_ref[...] = y


def _mm_bias(x, w, b, act="none", bm=1024):
    M, K = x.shape
    Nc = w.shape[1]
    if M <= bm:
        bm = M
    grid = (M // bm,)
    return pl.pallas_call(
        functools.partial(_mm_bias_kernel, act=act),
        grid=grid,
        in_specs=[
            pl.BlockSpec((bm, K), lambda i: (i, 0)),
            pl.BlockSpec((K, Nc), lambda i: (0, 0)),
            pl.BlockSpec((1, Nc), lambda i: (0, 0)),
        ],
        out_specs=pl.BlockSpec((bm, Nc), lambda i: (i, 0)),
        out_shape=jax.ShapeDtypeStruct((M, Nc), jnp.float32),
    )(x, w, b.reshape(1, Nc))


def _gru_math(gi, gh, h, H):
    r = jax.nn.sigmoid(gi[:, :H] + gh[:, :H])
    z = jax.nn.sigmoid(gi[:, H:2 * H] + gh[:, H:2 * H])
    n = jnp.tanh(gi[:, 2 * H:] + r * gh[:, 2 * H:])
    return (1.0 - z) * n + z * h


def _seq_scan_kernel(gif_ref, gib_ref, whf_ref, whb_ref, bhf_ref, bhb_ref,
                     o_ref, hf, hb, sumf, sumb, maxf, maxb):
    t = pl.program_id(0)

    @pl.when(t == 0)
    def _():
        hf[...] = jnp.zeros_like(hf)
        hb[...] = jnp.zeros_like(hb)
        sumf[...] = jnp.zeros_like(sumf)
        sumb[...] = jnp.zeros_like(sumb)
        maxf[...] = jnp.full_like(maxf, -jnp.inf)
        maxb[...] = jnp.full_like(maxb, -jnp.inf)

    ghf = jnp.dot(hf[...], whf_ref[...],
                  preferred_element_type=jnp.float32) + bhf_ref[...]
    hfn = _gru_math(gif_ref[0], ghf, hf[...], HS)
    hf[...] = hfn
    sumf[...] += hfn
    maxf[...] = jnp.maximum(maxf[...], hfn)

    ghb = jnp.dot(hb[...], whb_ref[...],
                  preferred_element_type=jnp.float32) + bhb_ref[...]
    hbn = _gru_math(gib_ref[0], ghb, hb[...], HS)
    hb[...] = hbn
    sumb[...] += hbn
    maxb[...] = jnp.maximum(maxb[...], hbn)

    @pl.when(t == T - 1)
    def _():
        o_ref[0] = sumf[...]
        o_ref[1] = sumb[...]
        o_ref[2] = maxf[...]
        o_ref[3] = maxb[...]


def _seq_branch(seq, p):
    x2d = seq.reshape(B * T, D)
    wf = jnp.concatenate([p['sWih_f'].T, p['sWih_b'].T], axis=1)
    bf = jnp.concatenate([p['sbih_f'], p['sbih_b']])
    gi = _mm_bias(x2d, wf, bf)
    gi = gi.reshape(B, T, 2, 3 * HS).transpose(2, 1, 0, 3)
    gif, gib = gi[0], gi[1]
    out = pl.pallas_call(
        _seq_scan_kernel,
        grid=(T,),
        in_specs=[
            pl.BlockSpec((1, B, 3 * HS), lambda t: (t, 0, 0)),
            pl.BlockSpec((1, B, 3 * HS), lambda t: (T - 1 - t, 0, 0)),
            pl.BlockSpec((HS, 3 * HS), lambda t: (0, 0)),
            pl.BlockSpec((HS, 3 * HS), lambda t: (0, 0)),
            pl.BlockSpec((1, 3 * HS), lambda t: (0, 0)),
            pl.BlockSpec((1, 3 * HS), lambda t: (0, 0)),
        ],
        out_specs=pl.BlockSpec((4, B, HS), lambda t: (0, 0, 0)),
        out_shape=jax.ShapeDtypeStruct((4, B, HS), jnp.float32),
        scratch_shapes=[pltpu.VMEM((B, HS), jnp.float32)] * 6,
    )(gif, gib, p['sWhh_f'].T, p['sWhh_b'].T,
      p['sbhh_f'].reshape(1, 3 * HS), p['sbhh_b'].reshape(1, 3 * HS))
    seq1 = (out[0] + out[1]) / T
    seq1 = jnp.concatenate([out[0], out[1]], axis=1) / T
    seq2 = jnp.concatenate([out[2], out[3]], axis=1)
    return seq1, seq2


def _graph_scan_kernel(gif_ref, gib_ref, whf_ref, whb_ref, bhf_ref, bhb_ref,
                       yf_ref, yb_ref, hf, hb):
    t = pl.program_id(0)

    @pl.when(t == 0)
    def _():
        hf[...] = jnp.zeros_like(hf)
        hb[...] = jnp.zeros_like(hb)

    ghf = jnp.dot(hf[...], whf_ref[...],
                  preferred_element_type=jnp.float32) + bhf_ref[...]
    hfn = _gru_math(gif_ref[0], ghf, hf[...], HG)
    hf[...] = hfn
    yf_ref[0] = hfn

    ghb = jnp.dot(hb[...], whb_ref[...],
                  preferred_element_type=jnp.float32) + bhb_ref[...]
    hbn = _gru_math(gib_ref[0], ghb, hb[...], HG)
    hb[...] = hbn
    yb_ref[0] = hbn


def _graph_branch(features, p):
    wf = jnp.concatenate([p['gWih_f'].T, p['gWih_b'].T], axis=1)
    bf = jnp.concatenate([p['gbih_f'], p['gbih_b']])
    gi = _mm_bias(features, wf, bf)
    gi = gi.reshape(B, NPG, 2, 3 * HG).transpose(2, 1, 0, 3)
    gif, gib = gi[0], gi[1]
    yf, yb = pl.pallas_call(
        _graph_scan_kernel,
        grid=(NPG,),
        in_specs=[
            pl.BlockSpec((1, B, 3 * HG), lambda t: (t, 0, 0)),
            pl.BlockSpec((1, B, 3 * HG), lambda t: (NPG - 1 - t, 0, 0)),
            pl.BlockSpec((HG, 3 * HG), lambda t: (0, 0)),
            pl.BlockSpec((HG, 3 * HG), lambda t: (0, 0)),
            pl.BlockSpec((1, 3 * HG), lambda t: (0, 0)),
            pl.BlockSpec((1, 3 * HG), lambda t: (0, 0)),
        ],
        out_specs=[
            pl.BlockSpec((1, B, HG), lambda t: (t, 0, 0)),
            pl.BlockSpec((1, B, HG), lambda t: (NPG - 1 - t, 0, 0)),
        ],
        out_shape=[
            jax.ShapeDtypeStruct((NPG, B, HG), jnp.float32),
            jax.ShapeDtypeStruct((NPG, B, HG), jnp.float32),
        ],
        scratch_shapes=[pltpu.VMEM((B, HG), jnp.float32)] * 2,
    )(gif, gib, p['gWhh_f'].T, p['gWhh_b'].T,
      p['gbhh_f'].reshape(1, 3 * HG), p['gbhh_b'].reshape(1, 3 * HG))
    st = jnp.concatenate([yf, yb], axis=2).transpose(1, 0, 2)
    return st.reshape(N, 2 * HG)


def _mlp_readout(x, W0, b0, W1, b1, W2, b2):
    h = _mm_bias(x, W0.T, b0, act="relu")
    h = _mm_bias(h, W1.T, b1, act="relu")
    return _mm_bias(h, W2.T, b2)


def kernel(features, edge_index, seq, params):
    p = params
    seq1, seq2 = _seq_branch(seq, p)
    h = _graph_branch(features, p)
    h = _appnp_sc(h, edge_index)
    stg = h.reshape(B, NPG, 2 * HG)
    st1 = jnp.max(stg, axis=1)
    st2 = jnp.mean(stg, axis=1)
    outputs = _mlp_readout(st1 + st2, p['mW0'], p['mb0'], p['mW1'], p['mb1'],
                           p['mW2'], p['mb2'])
    outputs1 = _mlp_readout(seq1 + seq2, p['nW0'], p['nb0'], p['nW1'], p['nb1'],
                            p['nW2'], p['nb2'])
    out = outputs1 + outputs
    return (out, out, out)


# speedup vs baseline: 1.0539x; 1.0539x over previous
import functools

import jax
import jax.numpy as jnp
from jax import lax
from jax.experimental import pallas as pl
from jax.experimental.pallas import tpu as pltpu
from jax.experimental.pallas import tpu_sc as plsc

N = 16384
E = 262144
B = 64
NPG = N // B
D = 128
HG = 128
HS = 512
T = 512
K_STEPS = 16
ALPHA = 0.1

NC = 2
NS = 16
L = 16

EPT = E // NS
SELF_PT = N // NS
KCH = 128
QROWS = (EPT + SELF_PT) // KCH
QCAP = QROWS * KCH
DH = 32
NP = 128 // DH

_mesh = plsc.VectorSubcoreMesh(core_axis_name="c", subcore_axis_name="s",
                               num_cores=NC, num_subcores=NS)


def _i32(x):
    return jnp.asarray(x, jnp.int32)


def _popcount16(mask):
    v = jnp.where(mask, 1, 0).astype(jnp.int32)
    pc = v[0]
    for k in range(1, L):
        pc = pc + v[k]
    return pc


def _sc_preprocess(edge_ref, qsa_ref, qd_ref, sbs, sbd):
    cid = lax.axis_index("c")
    sid = lax.axis_index("s")
    iot = lax.iota(jnp.int32, L)

    pltpu.sync_copy(edge_ref.at[0, pl.ds(sid * EPT, EPT)], sbs)
    pltpu.sync_copy(edge_ref.at[1, pl.ds(sid * EPT, EPT)], sbd)

    base = cid * NP * N

    def edge_body(i, _):
        s16 = sbs[pl.ds(i * L, L)]
        sbs[pl.ds(i * L, L)] = s16 + base
        return 0

    lax.fori_loop(0, EPT // L, edge_body, 0)
    pltpu.sync_copy(sbs, qsa_ref.at[cid, sid, pl.ds(0, EPT)])
    pltpu.sync_copy(sbd, qd_ref.at[cid, sid, pl.ds(0, EPT)])

    def self_body(i, _):
        node = sid * SELF_PT + i * L + iot
        sbs[pl.ds(i * L, L)] = node + base
        sbd[pl.ds(i * L, L)] = node
        return 0

    lax.fori_loop(0, SELF_PT // L, self_body, 0)
    pltpu.sync_copy(sbs.at[pl.ds(0, SELF_PT)],
                    qsa_ref.at[cid, sid, pl.ds(EPT, SELF_PT)])
    pltpu.sync_copy(sbd.at[pl.ds(0, SELF_PT)],
                    qd_ref.at[cid, sid, pl.ds(EPT, SELF_PT)])


def _sc_iterate(u_in_ref, qsa_ref, qd_ref, br_ref, cc_ref,
                u_out_ref, qsw_v, qd_v, gbuf, tacc, tcc, tbc, z64, acc_ref,
                gsem, ssem):
    cid = lax.axis_index("c")
    sid = lax.axis_index("s")

    pltpu.sync_copy(qsa_ref.at[cid, sid], qsw_v)
    pltpu.sync_copy(qd_ref.at[cid, sid], qd_v)

    def zb(r, _):
        for j in range(DH // L):
            z64[r, pl.ds(j * L, L)] = jnp.zeros((L,), jnp.float32)
        return 0

    lax.fori_loop(0, 64, zb, 0)

    def az(k, _):
        pltpu.sync_copy(z64, acc_ref.at[pl.ds(sid * 1024 + k * 64, 64)])
        return 0

    lax.fori_loop(0, 16, az, 0)
    plsc.subcore_barrier()

    for p in range(NP):
        if p > 0:
            def shift(r, _):
                for j in range(KCH // L):
                    sl = pl.ds(j * L, L)
                    qsw_v[r, sl] = qsw_v[r, sl] + N
                return 0

            lax.fori_loop(0, QROWS, shift, 0)

        def outer(jj, _):
            base = jj * 8
            gds = []
            for k in range(8):
                gds.append(pltpu.async_copy(
                    u_in_ref.at[qsw_v.at[base + k]], gbuf.at[k], gsem.at[k]))
            sds = []
            for k in range(8):
                gds[k].wait()
                sds.append(pltpu.async_copy(
                    gbuf.at[k], acc_ref.at[qd_v.at[base + k]], ssem.at[k],
                    add=True))
            for k in range(8):
                sds[k].wait()
            return 0

        lax.fori_loop(0, QROWS // 8, outer, 0)
        plsc.subcore_barrier()

        obase = (cid * NP + p) * N

        def blk(b, _):
            lbase = sid * 1024 + b * 64
            pltpu.sync_copy(acc_ref.at[pl.ds(lbase, 64)], tacc)
            pltpu.sync_copy(br_ref.at[pl.ds(lbase, 64)], tbc)
            pltpu.sync_copy(cc_ref.at[pl.ds(obase + lbase, 64)], tcc)
            pltpu.sync_copy(z64, acc_ref.at[pl.ds(lbase, 64)])

            def row(i, _):
                for q in range(DH // L):
                    sl = pl.ds(q * L, L)
                    tacc[i, sl] = tacc[i, sl] * tbc[i, sl] + tcc[i, sl]
                return 0

            lax.fori_loop(0, 64, row, 0)
            pltpu.sync_copy(tacc, u_out_ref.at[pl.ds(obase + lbase, 64)])
            return 0

        lax.fori_loop(0, 16, blk, 0)
        plsc.subcore_barrier()


_preprocess_call = functools.partial(
    pl.kernel,
    out_type=[
        jax.ShapeDtypeStruct((NC, NS, QCAP), jnp.int32),
        jax.ShapeDtypeStruct((NC, NS, QCAP), jnp.int32),
    ],
    mesh=_mesh,
    scratch_types=[
        pltpu.VMEM((EPT,), jnp.int32),
        pltpu.VMEM((EPT,), jnp.int32),
    ],
)


_iterate_call = functools.partial(
    pl.kernel,
    out_type=jax.ShapeDtypeStruct((2 * NP * N, DH), jnp.float32),
    mesh=_mesh,
    compiler_params=pltpu.CompilerParams(use_tc_tiling_on_sc=False),
    scratch_types=[
        pltpu.VMEM((QROWS, KCH), jnp.int32),
        pltpu.VMEM((QROWS, KCH), jnp.int32),
        pltpu.VMEM((8, KCH, DH), jnp.float32),
        pltpu.VMEM((64, DH), jnp.float32),
        pltpu.VMEM((64, DH), jnp.float32),
        pltpu.VMEM((64, DH), jnp.float32),
        pltpu.VMEM((64, DH), jnp.float32),
        pltpu.VMEM_SHARED((N, DH), jnp.float32),
        pltpu.SemaphoreType.DMA((8,)),
        pltpu.SemaphoreType.DMA((8,)),
    ],
)


def _appnp_sc(h, edge_index):
    qsa, qd = _preprocess_call(_sc_preprocess)(edge_index)
    qsa = qsa.reshape(NC, NS, QROWS, KCH)
    qd = qd.reshape(NC, NS, QROWS, KCH)
    it = _iterate_call(_sc_iterate)
    deg_rows = it(jnp.ones((2 * NP * N, DH), jnp.float32), qsa, qd,
                  jnp.ones((N, DH), jnp.float32),
                  jnp.zeros((2 * NP * N, DH), jnp.float32))
    deg = deg_rows[:N, 0]
    g = deg ** -0.5
    beta_rows = jnp.broadcast_to(((1.0 - ALPHA) / deg)[:, None], (N, DH))
    u = ((h * g[:, None]).reshape(N, 2, NP, DH)
         .transpose(1, 2, 0, 3).reshape(2 * NP * N, DH))
    cc = ALPHA * u
    for _ in range(K_STEPS):
        u = it(u, qsa, qd, beta_rows, cc)
    hf = (u.reshape(2, NP, N, DH).transpose(2, 0, 1, 3).reshape(N, 2 * HG)
          * jnp.sqrt(deg)[:, None])
    return hf


def _mm_bias_kernel(x_ref, w_ref, b_ref, o_ref, *, act):
    y = jnp.dot(x_ref[...], w_ref[...],
                preferred_element_type=jnp.float32) + b_ref[...]
    if act == "relu":
        y = jnp.maximum(y, 0.0)
    o_ref[...] = y


def _mm_bias(x, w, b, act="none", bm=1024):
    M, K = x.shape
    Nc = w.shape[1]
    if M <= bm:
        bm = M
    grid = (M // bm,)
    return pl.pallas_call(
        functools.partial(_mm_bias_kernel, act=act),
        grid=grid,
        in_specs=[
            pl.BlockSpec((bm, K), lambda i: (i, 0)),
            pl.BlockSpec((K, Nc), lambda i: (0, 0)),
            pl.BlockSpec((1, Nc), lambda i: (0, 0)),
        ],
        out_specs=pl.BlockSpec((bm, Nc), lambda i: (i, 0)),
        out_shape=jax.ShapeDtypeStruct((M, Nc), jnp.float32),
    )(x, w, b.reshape(1, Nc))


def _gru_math(gi, gh, h, H):
    r = jax.nn.sigmoid(gi[:, :H] + gh[:, :H])
    z = jax.nn.sigmoid(gi[:, H:2 * H] + gh[:, H:2 * H])
    n = jnp.tanh(gi[:, 2 * H:] + r * gh[:, 2 * H:])
    return (1.0 - z) * n + z * h


def _seq_scan_kernel(gif_ref, gib_ref, whf_ref, whb_ref, bhf_ref, bhb_ref,
                     o_ref, hf, hb, sumf, sumb, maxf, maxb):
    t = pl.program_id(0)

    @pl.when(t == 0)
    def _():
        hf[...] = jnp.zeros_like(hf)
        hb[...] = jnp.zeros_like(hb)
        sumf[...] = jnp.zeros_like(sumf)
        sumb[...] = jnp.zeros_like(sumb)
        maxf[...] = jnp.full_like(maxf, -jnp.inf)
        maxb[...] = jnp.full_like(maxb, -jnp.inf)

    ghf = jnp.dot(hf[...], whf_ref[...],
                  preferred_element_type=jnp.float32) + bhf_ref[...]
    hfn = _gru_math(gif_ref[0], ghf, hf[...], HS)
    hf[...] = hfn
    sumf[...] += hfn
    maxf[...] = jnp.maximum(maxf[...], hfn)

    ghb = jnp.dot(hb[...], whb_ref[...],
                  preferred_element_type=jnp.float32) + bhb_ref[...]
    hbn = _gru_math(gib_ref[0], ghb, hb[...], HS)
    hb[...] = hbn
    sumb[...] += hbn
    maxb[...] = jnp.maximum(maxb[...], hbn)

    @pl.when(t == T - 1)
    def _():
        o_ref[0] = sumf[...]
        o_ref[1] = sumb[...]
        o_ref[2] = maxf[...]
        o_ref[3] = maxb[...]


def _seq_branch(seq, p):
    x2d = seq.reshape(B * T, D)
    wf = jnp.concatenate([p['sWih_f'].T, p['sWih_b'].T], axis=1)
    bf = jnp.concatenate([p['sbih_f'], p['sbih_b']])
    gi = _mm_bias(x2d, wf, bf)
    gi = gi.reshape(B, T, 2, 3 * HS).transpose(2, 1, 0, 3)
    gif, gib = gi[0], gi[1]
    out = pl.pallas_call(
        _seq_scan_kernel,
        grid=(T,),
        in_specs=[
            pl.BlockSpec((1, B, 3 * HS), lambda t: (t, 0, 0)),
            pl.BlockSpec((1, B, 3 * HS), lambda t: (T - 1 - t, 0, 0)),
            pl.BlockSpec((HS, 3 * HS), lambda t: (0, 0)),
            pl.BlockSpec((HS, 3 * HS), lambda t: (0, 0)),
            pl.BlockSpec((1, 3 * HS), lambda t: (0, 0)),
            pl.BlockSpec((1, 3 * HS), lambda t: (0, 0)),
        ],
        out_specs=pl.BlockSpec((4, B, HS), lambda t: (0, 0, 0)),
        out_shape=jax.ShapeDtypeStruct((4, B, HS), jnp.float32),
        scratch_shapes=[pltpu.VMEM((B, HS), jnp.float32)] * 6,
    )(gif, gib, p['sWhh_f'].T, p['sWhh_b'].T,
      p['sbhh_f'].reshape(1, 3 * HS), p['sbhh_b'].reshape(1, 3 * HS))
    seq1 = jnp.concatenate([out[0], out[1]], axis=1) / T
    seq2 = jnp.concatenate([out[2], out[3]], axis=1)
    return seq1, seq2


def _graph_scan_kernel(gif_ref, gib_ref, whf_ref, whb_ref, bhf_ref, bhb_ref,
                       yf_ref, yb_ref, hf, hb):
    t = pl.program_id(0)

    @pl.when(t == 0)
    def _():
        hf[...] = jnp.zeros_like(hf)
        hb[...] = jnp.zeros_like(hb)

    ghf = jnp.dot(hf[...], whf_ref[...],
                  preferred_element_type=jnp.float32) + bhf_ref[...]
    hfn = _gru_math(gif_ref[0], ghf, hf[...], HG)
    hf[...] = hfn
    yf_ref[0] = hfn

    ghb = jnp.dot(hb[...], whb_ref[...],
                  preferred_element_type=jnp.float32) + bhb_ref[...]
    hbn = _gru_math(gib_ref[0], ghb, hb[...], HG)
    hb[...] = hbn
    yb_ref[0] = hbn


def _graph_branch(features, p):
    wf = jnp.concatenate([p['gWih_f'].T, p['gWih_b'].T], axis=1)
    bf = jnp.concatenate([p['gbih_f'], p['gbih_b']])
    gi = _mm_bias(features, wf, bf)
    gi = gi.reshape(B, NPG, 2, 3 * HG).transpose(2, 1, 0, 3)
    gif, gib = gi[0], gi[1]
    yf, yb = pl.pallas_call(
        _graph_scan_kernel,
        grid=(NPG,),
        in_specs=[
            pl.BlockSpec((1, B, 3 * HG), lambda t: (t, 0, 0)),
            pl.BlockSpec((1, B, 3 * HG), lambda t: (NPG - 1 - t, 0, 0)),
            pl.BlockSpec((HG, 3 * HG), lambda t: (0, 0)),
            pl.BlockSpec((HG, 3 * HG), lambda t: (0, 0)),
            pl.BlockSpec((1, 3 * HG), lambda t: (0, 0)),
            pl.BlockSpec((1, 3 * HG), lambda t: (0, 0)),
        ],
        out_specs=[
            pl.BlockSpec((1, B, HG), lambda t: (t, 0, 0)),
            pl.BlockSpec((1, B, HG), lambda t: (NPG - 1 - t, 0, 0)),
        ],
        out_shape=[
            jax.ShapeDtypeStruct((NPG, B, HG), jnp.float32),
            jax.ShapeDtypeStruct((NPG, B, HG), jnp.float32),
        ],
        scratch_shapes=[pltpu.VMEM((B, HG), jnp.float32)] * 2,
    )(gif, gib, p['gWhh_f'].T, p['gWhh_b'].T,
      p['gbhh_f'].reshape(1, 3 * HG), p['gbhh_b'].reshape(1, 3 * HG))
    st = jnp.concatenate([yf, yb], axis=2).transpose(1, 0, 2)
    return st.reshape(N, 2 * HG)


def _mlp_readout(x, W0, b0, W1, b1, W2, b2):
    h = _mm_bias(x, W0.T, b0, act="relu")
    h = _mm_bias(h, W1.T, b1, act="relu")
    return _mm_bias(h, W2.T, b2)


def kernel(features, edge_index, seq, params):
    p = params
    seq1, seq2 = _seq_branch(seq, p)
    h = _graph_branch(features, p)
    h = _appnp_sc(h, edge_index)
    stg = h.reshape(B, NPG, 2 * HG)
    st1 = jnp.max(stg, axis=1)
    st2 = jnp.mean(stg, axis=1)
    outputs = _mlp_readout(st1 + st2, p['mW0'], p['mb0'], p['mW1'], p['mb1'],
                           p['mW2'], p['mb2'])
    outputs1 = _mlp_readout(seq1 + seq2, p['nW0'], p['nb0'], p['nW1'], p['nb1'],
                            p['nW2'], p['nb2'])
    out = outputs1 + outputs
    return (out, out, out)


# speedup vs baseline: 1.0541x; 1.0002x over previous
import functools

import jax
import jax.numpy as jnp
from jax import lax
from jax.experimental import pallas as pl
from jax.experimental.pallas import tpu as pltpu
from jax.experimental.pallas import tpu_sc as plsc

N = 16384
E = 262144
B = 64
NPG = N // B
D = 128
HG = 128
HS = 512
T = 512
K_STEPS = 16
ALPHA = 0.1

NC = 2
NS = 16
L = 16

EPT = E // NS
SELF_PT = N // NS
KCH = 128
QROWS = (EPT + SELF_PT) // KCH
QCAP = QROWS * KCH
DH = 32
NP = 128 // DH

_mesh = plsc.VectorSubcoreMesh(core_axis_name="c", subcore_axis_name="s",
                               num_cores=NC, num_subcores=NS)


def _sc_preprocess(edge_ref, qsa_ref, qd_ref, sbs, sbd):
    cid = lax.axis_index("c")
    sid = lax.axis_index("s")
    iot = lax.iota(jnp.int32, L)

    pltpu.sync_copy(edge_ref.at[0, pl.ds(sid * EPT, EPT)], sbs)
    pltpu.sync_copy(edge_ref.at[1, pl.ds(sid * EPT, EPT)], sbd)

    base = cid * NP * N

    def edge_body(i, _):
        s16 = sbs[pl.ds(i * L, L)]
        sbs[pl.ds(i * L, L)] = s16 + base
        return 0

    lax.fori_loop(0, EPT // L, edge_body, 0)
    pltpu.sync_copy(sbs, qsa_ref.at[cid, sid, pl.ds(0, EPT)])
    pltpu.sync_copy(sbd, qd_ref.at[cid, sid, pl.ds(0, EPT)])

    def self_body(i, _):
        node = sid * SELF_PT + i * L + iot
        sbs[pl.ds(i * L, L)] = node + base
        sbd[pl.ds(i * L, L)] = node
        return 0

    lax.fori_loop(0, SELF_PT // L, self_body, 0)
    pltpu.sync_copy(sbs.at[pl.ds(0, SELF_PT)],
                    qsa_ref.at[cid, sid, pl.ds(EPT, SELF_PT)])
    pltpu.sync_copy(sbd.at[pl.ds(0, SELF_PT)],
                    qd_ref.at[cid, sid, pl.ds(EPT, SELF_PT)])


def _sc_iterate(u_in_ref, qsa_ref, qd_ref, br_ref, cc_ref,
                u_out_ref, qsw_v, qd_v, gbuf, tacc, tcc, tbc, z64, acc_ref,
                gsem, ssem):
    cid = lax.axis_index("c")
    sid = lax.axis_index("s")

    pltpu.sync_copy(qsa_ref.at[cid, sid], qsw_v)
    pltpu.sync_copy(qd_ref.at[cid, sid], qd_v)

    def zb(r, _):
        for j in range(DH // L):
            z64[r, pl.ds(j * L, L)] = jnp.zeros((L,), jnp.float32)
        return 0

    lax.fori_loop(0, 64, zb, 0)

    def az(k, _):
        pltpu.sync_copy(z64, acc_ref.at[pl.ds(sid * 1024 + k * 64, 64)])
        return 0

    lax.fori_loop(0, 16, az, 0)
    plsc.subcore_barrier()

    for p in range(NP):
        if p > 0:
            def shift(r, _):
                for j in range(KCH // L):
                    sl = pl.ds(j * L, L)
                    qsw_v[r, sl] = qsw_v[r, sl] + N
                return 0

            lax.fori_loop(0, QROWS, shift, 0)

        def outer(jj, _):
            base = jj * 8
            gds = []
            for k in range(8):
                gds.append(pltpu.async_copy(
                    u_in_ref.at[qsw_v.at[base + k]], gbuf.at[k], gsem.at[k]))
            sds = []
            for k in range(8):
                gds[k].wait()
                sds.append(pltpu.async_copy(
                    gbuf.at[k], acc_ref.at[qd_v.at[base + k]], ssem.at[k],
                    add=True))
            for k in range(8):
                sds[k].wait()
            return 0

        lax.fori_loop(0, QROWS // 8, outer, 0)
        plsc.subcore_barrier()

        obase = (cid * NP + p) * N

        def blk(b, _):
            lbase = sid * 1024 + b * 64
            pltpu.sync_copy(acc_ref.at[pl.ds(lbase, 64)], tacc)
            pltpu.sync_copy(br_ref.at[pl.ds(lbase, 64)], tbc)
            pltpu.sync_copy(cc_ref.at[pl.ds(obase + lbase, 64)], tcc)
            pltpu.sync_copy(z64, acc_ref.at[pl.ds(lbase, 64)])

            def row(i, _):
                for q in range(DH // L):
                    sl = pl.ds(q * L, L)
                    tacc[i, sl] = tacc[i, sl] * tbc[i, sl] + tcc[i, sl]
                return 0

            lax.fori_loop(0, 64, row, 0)
            pltpu.sync_copy(tacc, u_out_ref.at[pl.ds(obase + lbase, 64)])
            return 0

        lax.fori_loop(0, 16, blk, 0)
        plsc.subcore_barrier()


_preprocess_call = functools.partial(
    pl.kernel,
    out_type=[
        jax.ShapeDtypeStruct((NC, NS, QCAP), jnp.int32),
        jax.ShapeDtypeStruct((NC, NS, QCAP), jnp.int32),
    ],
    mesh=_mesh,
    scratch_types=[
        pltpu.VMEM((EPT,), jnp.int32),
        pltpu.VMEM((EPT,), jnp.int32),
    ],
)


_iterate_call = functools.partial(
    pl.kernel,
    out_type=jax.ShapeDtypeStruct((2 * NP * N, DH), jnp.float32),
    mesh=_mesh,
    compiler_params=pltpu.CompilerParams(use_tc_tiling_on_sc=False),
    scratch_types=[
        pltpu.VMEM((QROWS, KCH), jnp.int32),
        pltpu.VMEM((QROWS, KCH), jnp.int32),
        pltpu.VMEM((8, KCH, DH), jnp.float32),
        pltpu.VMEM((64, DH), jnp.float32),
        pltpu.VMEM((64, DH), jnp.float32),
        pltpu.VMEM((64, DH), jnp.float32),
        pltpu.VMEM((64, DH), jnp.float32),
        pltpu.VMEM_SHARED((N, DH), jnp.float32),
        pltpu.SemaphoreType.DMA((8,)),
        pltpu.SemaphoreType.DMA((8,)),
    ],
)


def _appnp_sc(h, edge_index):
    qsa, qd = _preprocess_call(_sc_preprocess)(edge_index)
    qsa = qsa.reshape(NC, NS, QROWS, KCH)
    qd = qd.reshape(NC, NS, QROWS, KCH)
    it = _iterate_call(_sc_iterate)
    deg_rows = it(jnp.ones((2 * NP * N, DH), jnp.float32), qsa, qd,
                  jnp.ones((N, DH), jnp.float32),
                  jnp.zeros((2 * NP * N, DH), jnp.float32))
    deg = deg_rows[:N, 0]
    g = deg ** -0.5
    beta_rows = jnp.broadcast_to(((1.0 - ALPHA) / deg)[:, None], (N, DH))
    u = ((h * g[:, None]).reshape(N, 2, NP, DH)
         .transpose(1, 2, 0, 3).reshape(2 * NP * N, DH))
    cc = ALPHA * u
    for _ in range(K_STEPS):
        u = it(u, qsa, qd, beta_rows, cc)
    hf = (u.reshape(2, NP, N, DH).transpose(2, 0, 1, 3).reshape(N, 2 * HG)
          * jnp.sqrt(deg)[:, None])
    return hf


def _mm_bias_kernel(x_ref, w_ref, b_ref, o_ref, *, act):
    y = jnp.dot(x_ref[...], w_ref[...],
                preferred_element_type=jnp.float32) + b_ref[...]
    if act == "relu":
        y = jnp.maximum(y, 0.0)
    o_ref[...] = y


def _mm_bias(x, w, b, act="none", bm=1024):
    M, K = x.shape
    Nc = w.shape[1]
    if M <= bm:
        bm = M
    grid = (M // bm,)
    return pl.pallas_call(
        functools.partial(_mm_bias_kernel, act=act),
        grid=grid,
        in_specs=[
            pl.BlockSpec((bm, K), lambda i: (i, 0)),
            pl.BlockSpec((K, Nc), lambda i: (0, 0)),
            pl.BlockSpec((1, Nc), lambda i: (0, 0)),
        ],
        out_specs=pl.BlockSpec((bm, Nc), lambda i: (i, 0)),
        out_shape=jax.ShapeDtypeStruct((M, Nc), jnp.float32),
    )(x, w, b.reshape(1, Nc))


def _gru_math(gi, gh, h, H):
    r = jax.nn.sigmoid(gi[:, :H] + gh[:, :H])
    z = jax.nn.sigmoid(gi[:, H:2 * H] + gh[:, H:2 * H])
    n = jnp.tanh(gi[:, 2 * H:] + r * gh[:, 2 * H:])
    return (1.0 - z) * n + z * h


def _seq_scan_kernel(gif_ref, gib_ref, whf_ref, whb_ref, bhf_ref, bhb_ref,
                     o_ref, hf, hb, sumf, sumb, maxf, maxb):
    t = pl.program_id(0)

    @pl.when(t == 0)
    def _():
        hf[...] = jnp.zeros_like(hf)
        hb[...] = jnp.zeros_like(hb)
        sumf[...] = jnp.zeros_like(sumf)
        sumb[...] = jnp.zeros_like(sumb)
        maxf[...] = jnp.full_like(maxf, -jnp.inf)
        maxb[...] = jnp.full_like(maxb, -jnp.inf)

    ghf = jnp.dot(hf[...], whf_ref[...],
                  preferred_element_type=jnp.float32) + bhf_ref[...]
    hfn = _gru_math(gif_ref[0], ghf, hf[...], HS)
    hf[...] = hfn
    sumf[...] += hfn
    maxf[...] = jnp.maximum(maxf[...], hfn)

    ghb = jnp.dot(hb[...], whb_ref[...],
                  preferred_element_type=jnp.float32) + bhb_ref[...]
    hbn = _gru_math(gib_ref[0], ghb, hb[...], HS)
    hb[...] = hbn
    sumb[...] += hbn
    maxb[...] = jnp.maximum(maxb[...], hbn)

    @pl.when(t == T - 1)
    def _():
        o_ref[0] = sumf[...]
        o_ref[1] = sumb[...]
        o_ref[2] = maxf[...]
        o_ref[3] = maxb[...]


def _seq_branch(seq, p):
    x2d = seq.reshape(B * T, D)
    wf = jnp.concatenate([p['sWih_f'].T, p['sWih_b'].T], axis=1)
    bf = jnp.concatenate([p['sbih_f'], p['sbih_b']])
    gi = _mm_bias(x2d, wf, bf)
    gi = gi.reshape(B, T, 2, 3 * HS).transpose(2, 1, 0, 3)
    gif, gib = gi[0], gi[1]
    out = pl.pallas_call(
        _seq_scan_kernel,
        grid=(T,),
        in_specs=[
            pl.BlockSpec((1, B, 3 * HS), lambda t: (t, 0, 0)),
            pl.BlockSpec((1, B, 3 * HS), lambda t: (T - 1 - t, 0, 0)),
            pl.BlockSpec((HS, 3 * HS), lambda t: (0, 0)),
            pl.BlockSpec((HS, 3 * HS), lambda t: (0, 0)),
            pl.BlockSpec((1, 3 * HS), lambda t: (0, 0)),
            pl.BlockSpec((1, 3 * HS), lambda t: (0, 0)),
        ],
        out_specs=pl.BlockSpec((4, B, HS), lambda t: (0, 0, 0)),
        out_shape=jax.ShapeDtypeStruct((4, B, HS), jnp.float32),
        scratch_shapes=[pltpu.VMEM((B, HS), jnp.float32)] * 6,
    )(gif, gib, p['sWhh_f'].T, p['sWhh_b'].T,
      p['sbhh_f'].reshape(1, 3 * HS), p['sbhh_b'].reshape(1, 3 * HS))
    seq1 = jnp.concatenate([out[0], out[1]], axis=1) / T
    seq2 = jnp.concatenate([out[2], out[3]], axis=1)
    return seq1, seq2


def _graph_scan_kernel(gif_ref, gib_ref, whf_ref, whb_ref, bhf_ref, bhb_ref,
                       yf_ref, yb_ref, hf, hb):
    t = pl.program_id(0)

    @pl.when(t == 0)
    def _():
        hf[...] = jnp.zeros_like(hf)
        hb[...] = jnp.zeros_like(hb)

    ghf = jnp.dot(hf[...], whf_ref[...],
                  preferred_element_type=jnp.float32) + bhf_ref[...]
    hfn = _gru_math(gif_ref[0], ghf, hf[...], HG)
    hf[...] = hfn
    yf_ref[0] = hfn

    ghb = jnp.dot(hb[...], whb_ref[...],
                  preferred_element_type=jnp.float32) + bhb_ref[...]
    hbn = _gru_math(gib_ref[0], ghb, hb[...], HG)
    hb[...] = hbn
    yb_ref[0] = hbn


def _graph_branch(features, p):
    wf = jnp.concatenate([p['gWih_f'].T, p['gWih_b'].T], axis=1)
    bf = jnp.concatenate([p['gbih_f'], p['gbih_b']])
    gi = _mm_bias(features, wf, bf)
    gi = gi.reshape(B, NPG, 2, 3 * HG).transpose(2, 1, 0, 3)
    gif, gib = gi[0], gi[1]
    yf, yb = pl.pallas_call(
        _graph_scan_kernel,
        grid=(NPG,),
        in_specs=[
            pl.BlockSpec((1, B, 3 * HG), lambda t: (t, 0, 0)),
            pl.BlockSpec((1, B, 3 * HG), lambda t: (NPG - 1 - t, 0, 0)),
            pl.BlockSpec((HG, 3 * HG), lambda t: (0, 0)),
            pl.BlockSpec((HG, 3 * HG), lambda t: (0, 0)),
            pl.BlockSpec((1, 3 * HG), lambda t: (0, 0)),
            pl.BlockSpec((1, 3 * HG), lambda t: (0, 0)),
        ],
        out_specs=[
            pl.BlockSpec((1, B, HG), lambda t: (t, 0, 0)),
            pl.BlockSpec((1, B, HG), lambda t: (NPG - 1 - t, 0, 0)),
        ],
        out_shape=[
            jax.ShapeDtypeStruct((NPG, B, HG), jnp.float32),
            jax.ShapeDtypeStruct((NPG, B, HG), jnp.float32),
        ],
        scratch_shapes=[pltpu.VMEM((B, HG), jnp.float32)] * 2,
    )(gif, gib, p['gWhh_f'].T, p['gWhh_b'].T,
      p['gbhh_f'].reshape(1, 3 * HG), p['gbhh_b'].reshape(1, 3 * HG))
    st = jnp.concatenate([yf, yb], axis=2).transpose(1, 0, 2)
    return st.reshape(N, 2 * HG)


def _mlp_readout(x, W0, b0, W1, b1, W2, b2):
    h = _mm_bias(x, W0.T, b0, act="relu")
    h = _mm_bias(h, W1.T, b1, act="relu")
    return _mm_bias(h, W2.T, b2)


def kernel(features, edge_index, seq, params):
    p = params
    seq1, seq2 = _seq_branch(seq, p)
    h = _graph_branch(features, p)
    h = _appnp_sc(h, edge_index)
    stg = h.reshape(B, NPG, 2 * HG)
    st1 = jnp.max(stg, axis=1)
    st2 = jnp.mean(stg, axis=1)
    outputs = _mlp_readout(st1 + st2, p['mW0'], p['mb0'], p['mW1'], p['mb1'],
                           p['mW2'], p['mb2'])
    outputs1 = _mlp_readout(seq1 + seq2, p['nW0'], p['nb0'], p['nW1'], p['nb1'],
                            p['nW2'], p['nb2'])
    out = outputs1 + outputs
    return (out, out, out)


# speedup vs baseline: 1.2075x; 1.1455x over previous
import functools

import jax
import jax.numpy as jnp
from jax import lax
from jax.experimental import pallas as pl
from jax.experimental.pallas import tpu as pltpu
from jax.experimental.pallas import tpu_sc as plsc

N = 16384
E = 262144
B = 64
NPG = N // B
D = 128
HG = 128
HS = 512
T = 512
K_STEPS = 16
ALPHA = 0.1

NC = 2
NS = 16
L = 16

EPT = E // NS
SELF_PT = N // NS
KCH = 128
QROWS = (EPT + SELF_PT) // KCH
QCAP = QROWS * KCH
DH = 32
NP = 128 // DH

_mesh = plsc.VectorSubcoreMesh(core_axis_name="c", subcore_axis_name="s",
                               num_cores=NC, num_subcores=NS)


def _sc_preprocess(edge_ref, qsa_ref, qd_ref, sbs, sbd):
    cid = lax.axis_index("c")
    sid = lax.axis_index("s")
    iot = lax.iota(jnp.int32, L)

    pltpu.sync_copy(edge_ref.at[0, pl.ds(sid * EPT, EPT)], sbs)
    pltpu.sync_copy(edge_ref.at[1, pl.ds(sid * EPT, EPT)], sbd)

    base = cid * NP * N

    def edge_body(i, _):
        s16 = sbs[pl.ds(i * L, L)]
        sbs[pl.ds(i * L, L)] = s16 + base
        return 0

    lax.fori_loop(0, EPT // L, edge_body, 0)
    pltpu.sync_copy(sbs, qsa_ref.at[cid, sid, pl.ds(0, EPT)])
    pltpu.sync_copy(sbd, qd_ref.at[cid, sid, pl.ds(0, EPT)])

    def self_body(i, _):
        node = sid * SELF_PT + i * L + iot
        sbs[pl.ds(i * L, L)] = node + base
        sbd[pl.ds(i * L, L)] = node
        return 0

    lax.fori_loop(0, SELF_PT // L, self_body, 0)
    pltpu.sync_copy(sbs.at[pl.ds(0, SELF_PT)],
                    qsa_ref.at[cid, sid, pl.ds(EPT, SELF_PT)])
    pltpu.sync_copy(sbd.at[pl.ds(0, SELF_PT)],
                    qd_ref.at[cid, sid, pl.ds(EPT, SELF_PT)])


def _sc_iterate(u_in_ref, qsa_ref, qd_ref, br_ref, cc_ref,
                u_out_ref, qsw_v, qd_v, gbuf, tacc, tcc, tbc, z64, acc_ref,
                gsem, ssem):
    cid = lax.axis_index("c")
    sid = lax.axis_index("s")

    pltpu.sync_copy(qsa_ref.at[cid, sid], qsw_v)
    pltpu.sync_copy(qd_ref.at[cid, sid], qd_v)

    def zb(r, _):
        for j in range(DH // L):
            z64[r, pl.ds(j * L, L)] = jnp.zeros((L,), jnp.float32)
        return 0

    lax.fori_loop(0, 64, zb, 0)

    def az(k, _):
        pltpu.sync_copy(z64, acc_ref.at[pl.ds(sid * 1024 + k * 64, 64)])
        return 0

    lax.fori_loop(0, 16, az, 0)
    plsc.subcore_barrier()

    for p in range(NP):
        if p > 0:
            def shift(r, _):
                for j in range(KCH // L):
                    sl = pl.ds(j * L, L)
                    qsw_v[r, sl] = qsw_v[r, sl] + N
                return 0

            lax.fori_loop(0, QROWS, shift, 0)

        def outer(jj, _):
            base = jj * 8
            gds = []
            for k in range(8):
                gds.append(pltpu.async_copy(
                    u_in_ref.at[qsw_v.at[base + k]], gbuf.at[k], gsem.at[k]))
            sds = []
            for k in range(8):
                gds[k].wait()
                sds.append(pltpu.async_copy(
                    gbuf.at[k], acc_ref.at[qd_v.at[base + k]], ssem.at[k],
                    add=True))
            for k in range(8):
                sds[k].wait()
            return 0

        lax.fori_loop(0, QROWS // 8, outer, 0)
        plsc.subcore_barrier()

        obase = (cid * NP + p) * N

        def blk(b, _):
            lbase = sid * 1024 + b * 128
            d1 = pltpu.async_copy(acc_ref.at[pl.ds(lbase, 128)], tacc,
                                  gsem.at[0])
            d2 = pltpu.async_copy(br_ref.at[pl.ds(lbase, 128)], tbc,
                                  gsem.at[1])
            d3 = pltpu.async_copy(cc_ref.at[pl.ds(obase + lbase, 128)], tcc,
                                  gsem.at[2])
            d1.wait()
            d2.wait()
            d3.wait()
            pltpu.sync_copy(z64, acc_ref.at[pl.ds(lbase, 64)])
            pltpu.sync_copy(z64, acc_ref.at[pl.ds(lbase + 64, 64)])

            def row(i, _):
                for q in range(DH // L):
                    sl = pl.ds(q * L, L)
                    tacc[i, sl] = tacc[i, sl] * tbc[i, sl] + tcc[i, sl]
                return 0

            lax.fori_loop(0, 128, row, 0)
            pltpu.sync_copy(tacc, u_out_ref.at[pl.ds(obase + lbase, 128)])
            return 0

        lax.fori_loop(0, 8, blk, 0)
        plsc.subcore_barrier()


_preprocess_call = functools.partial(
    pl.kernel,
    out_type=[
        jax.ShapeDtypeStruct((NC, NS, QCAP), jnp.int32),
        jax.ShapeDtypeStruct((NC, NS, QCAP), jnp.int32),
    ],
    mesh=_mesh,
    scratch_types=[
        pltpu.VMEM((EPT,), jnp.int32),
        pltpu.VMEM((EPT,), jnp.int32),
    ],
)


_iterate_call = functools.partial(
    pl.kernel,
    out_type=jax.ShapeDtypeStruct((2 * NP * N, DH), jnp.float32),
    mesh=_mesh,
    compiler_params=pltpu.CompilerParams(use_tc_tiling_on_sc=False),
    scratch_types=[
        pltpu.VMEM((QROWS, KCH), jnp.int32),
        pltpu.VMEM((QROWS, KCH), jnp.int32),
        pltpu.VMEM((8, KCH, DH), jnp.float32),
        pltpu.VMEM((128, DH), jnp.float32),
        pltpu.VMEM((128, DH), jnp.float32),
        pltpu.VMEM((128, DH), jnp.float32),
        pltpu.VMEM((64, DH), jnp.float32),
        pltpu.VMEM_SHARED((N, DH), jnp.float32),
        pltpu.SemaphoreType.DMA((8,)),
        pltpu.SemaphoreType.DMA((8,)),
    ],
)


def _appnp_sc(h, edge_index):
    qsa, qd = _preprocess_call(_sc_preprocess)(edge_index)
    qsa = qsa.reshape(NC, NS, QROWS, KCH)
    qd = qd.reshape(NC, NS, QROWS, KCH)
    it = _iterate_call(_sc_iterate)
    deg_rows = it(jnp.ones((2 * NP * N, DH), jnp.float32), qsa, qd,
                  jnp.ones((N, DH), jnp.float32),
                  jnp.zeros((2 * NP * N, DH), jnp.float32))
    deg = deg_rows[:N, 0]
    g = deg ** -0.5
    beta_rows = jnp.broadcast_to(((1.0 - ALPHA) / deg)[:, None], (N, DH))
    u = ((h * g[:, None]).reshape(N, 2, NP, DH)
         .transpose(1, 2, 0, 3).reshape(2 * NP * N, DH))
    cc = ALPHA * u
    for _ in range(K_STEPS):
        u = it(u, qsa, qd, beta_rows, cc)
    hf = (u.reshape(2, NP, N, DH).transpose(2, 0, 1, 3).reshape(N, 2 * HG)
          * jnp.sqrt(deg)[:, None])
    return hf


def _mm_bias_kernel(x_ref, w_ref, b_ref, o_ref, *, act):
    y = jnp.dot(x_ref[...], w_ref[...],
                preferred_element_type=jnp.float32) + b_ref[...]
    if act == "relu":
        y = jnp.maximum(y, 0.0)
    o_ref[...] = y


def _mm_bias(x, w, b, act="none", bm=1024):
    M, K = x.shape
    Nc = w.shape[1]
    if M <= bm:
        bm = M
    grid = (M // bm,)
    return pl.pallas_call(
        functools.partial(_mm_bias_kernel, act=act),
        grid=grid,
        in_specs=[
            pl.BlockSpec((bm, K), lambda i: (i, 0)),
            pl.BlockSpec((K, Nc), lambda i: (0, 0)),
            pl.BlockSpec((1, Nc), lambda i: (0, 0)),
        ],
        out_specs=pl.BlockSpec((bm, Nc), lambda i: (i, 0)),
        out_shape=jax.ShapeDtypeStruct((M, Nc), jnp.float32),
    )(x, w, b.reshape(1, Nc))


def _gru_math(gi, gh, h, H):
    r = jax.nn.sigmoid(gi[:, :H] + gh[:, :H])
    z = jax.nn.sigmoid(gi[:, H:2 * H] + gh[:, H:2 * H])
    n = jnp.tanh(gi[:, 2 * H:] + r * gh[:, 2 * H:])
    return (1.0 - z) * n + z * h


def _seq_scan_kernel(gif_ref, gib_ref, whf_ref, whb_ref, bhf_ref, bhb_ref,
                     o_ref, hf, hb, sumf, sumb, maxf, maxb):
    t = pl.program_id(0)

    @pl.when(t == 0)
    def _():
        hf[...] = jnp.zeros_like(hf)
        hb[...] = jnp.zeros_like(hb)
        sumf[...] = jnp.zeros_like(sumf)
        sumb[...] = jnp.zeros_like(sumb)
        maxf[...] = jnp.full_like(maxf, -jnp.inf)
        maxb[...] = jnp.full_like(maxb, -jnp.inf)

    ghf = jnp.dot(hf[...], whf_ref[...],
                  preferred_element_type=jnp.float32) + bhf_ref[...]
    hfn = _gru_math(gif_ref[0], ghf, hf[...], HS)
    hf[...] = hfn
    sumf[...] += hfn
    maxf[...] = jnp.maximum(maxf[...], hfn)

    ghb = jnp.dot(hb[...], whb_ref[...],
                  preferred_element_type=jnp.float32) + bhb_ref[...]
    hbn = _gru_math(gib_ref[0], ghb, hb[...], HS)
    hb[...] = hbn
    sumb[...] += hbn
    maxb[...] = jnp.maximum(maxb[...], hbn)

    @pl.when(t == T - 1)
    def _():
        o_ref[0] = sumf[...]
        o_ref[1] = sumb[...]
        o_ref[2] = maxf[...]
        o_ref[3] = maxb[...]


def _seq_branch(seq, p):
    x2d = seq.reshape(B * T, D)
    wf = jnp.concatenate([p['sWih_f'].T, p['sWih_b'].T], axis=1)
    bf = jnp.concatenate([p['sbih_f'], p['sbih_b']])
    gi = _mm_bias(x2d, wf, bf)
    gi = gi.reshape(B, T, 2, 3 * HS).transpose(2, 1, 0, 3)
    gif, gib = gi[0], gi[1]
    out = pl.pallas_call(
        _seq_scan_kernel,
        grid=(T,),
        in_specs=[
            pl.BlockSpec((1, B, 3 * HS), lambda t: (t, 0, 0)),
            pl.BlockSpec((1, B, 3 * HS), lambda t: (T - 1 - t, 0, 0)),
            pl.BlockSpec((HS, 3 * HS), lambda t: (0, 0)),
            pl.BlockSpec((HS, 3 * HS), lambda t: (0, 0)),
            pl.BlockSpec((1, 3 * HS), lambda t: (0, 0)),
            pl.BlockSpec((1, 3 * HS), lambda t: (0, 0)),
        ],
        out_specs=pl.BlockSpec((4, B, HS), lambda t: (0, 0, 0)),
        out_shape=jax.ShapeDtypeStruct((4, B, HS), jnp.float32),
        scratch_shapes=[pltpu.VMEM((B, HS), jnp.float32)] * 6,
    )(gif, gib, p['sWhh_f'].T, p['sWhh_b'].T,
      p['sbhh_f'].reshape(1, 3 * HS), p['sbhh_b'].reshape(1, 3 * HS))
    seq1 = jnp.concatenate([out[0], out[1]], axis=1) / T
    seq2 = jnp.concatenate([out[2], out[3]], axis=1)
    return seq1, seq2


def _graph_scan_kernel(gif_ref, gib_ref, whf_ref, whb_ref, bhf_ref, bhb_ref,
                       yf_ref, yb_ref, hf, hb):
    t = pl.program_id(0)

    @pl.when(t == 0)
    def _():
        hf[...] = jnp.zeros_like(hf)
        hb[...] = jnp.zeros_like(hb)

    ghf = jnp.dot(hf[...], whf_ref[...],
                  preferred_element_type=jnp.float32) + bhf_ref[...]
    hfn = _gru_math(gif_ref[0], ghf, hf[...], HG)
    hf[...] = hfn
    yf_ref[0] = hfn

    ghb = jnp.dot(hb[...], whb_ref[...],
                  preferred_element_type=jnp.float32) + bhb_ref[...]
    hbn = _gru_math(gib_ref[0], ghb, hb[...], HG)
    hb[...] = hbn
    yb_ref[0] = hbn


def _graph_branch(features, p):
    wf = jnp.concatenate([p['gWih_f'].T, p['gWih_b'].T], axis=1)
    bf = jnp.concatenate([p['gbih_f'], p['gbih_b']])
    gi = _mm_bias(features, wf, bf)
    gi = gi.reshape(B, NPG, 2, 3 * HG).transpose(2, 1, 0, 3)
    gif, gib = gi[0], gi[1]
    yf, yb = pl.pallas_call(
        _graph_scan_kernel,
        grid=(NPG,),
        in_specs=[
            pl.BlockSpec((1, B, 3 * HG), lambda t: (t, 0, 0)),
            pl.BlockSpec((1, B, 3 * HG), lambda t: (NPG - 1 - t, 0, 0)),
            pl.BlockSpec((HG, 3 * HG), lambda t: (0, 0)),
            pl.BlockSpec((HG, 3 * HG), lambda t: (0, 0)),
            pl.BlockSpec((1, 3 * HG), lambda t: (0, 0)),
            pl.BlockSpec((1, 3 * HG), lambda t: (0, 0)),
        ],
        out_specs=[
            pl.BlockSpec((1, B, HG), lambda t: (t, 0, 0)),
            pl.BlockSpec((1, B, HG), lambda t: (NPG - 1 - t, 0, 0)),
        ],
        out_shape=[
            jax.ShapeDtypeStruct((NPG, B, HG), jnp.float32),
            jax.ShapeDtypeStruct((NPG, B, HG), jnp.float32),
        ],
        scratch_shapes=[pltpu.VMEM((B, HG), jnp.float32)] * 2,
    )(gif, gib, p['gWhh_f'].T, p['gWhh_b'].T,
      p['gbhh_f'].reshape(1, 3 * HG), p['gbhh_b'].reshape(1, 3 * HG))
    st = jnp.concatenate([yf, yb], axis=2).transpose(1, 0, 2)
    return st.reshape(N, 2 * HG)


def _mlp_readout(x, W0, b0, W1, b1, W2, b2):
    h = _mm_bias(x, W0.T, b0, act="relu")
    h = _mm_bias(h, W1.T, b1, act="relu")
    return _mm_bias(h, W2.T, b2)


def kernel(features, edge_index, seq, params):
    p = params
    seq1, seq2 = _seq_branch(seq, p)
    h = _graph_branch(features, p)
    h = _appnp_sc(h, edge_index)
    stg = h.reshape(B, NPG, 2 * HG)
    st1 = jnp.max(stg, axis=1)
    st2 = jnp.mean(stg, axis=1)
    outputs = _mlp_readout(st1 + st2, p['mW0'], p['mb0'], p['mW1'], p['mb1'],
                           p['mW2'], p['mb2'])
    outputs1 = _mlp_readout(seq1 + seq2, p['nW0'], p['nb0'], p['nW1'], p['nb1'],
                            p['nW2'], p['nb2'])
    out = outputs1 + outputs
    return (out, out, out)


# speedup vs baseline: 1.2083x; 1.0007x over previous
import functools

import jax
import jax.numpy as jnp
from jax import lax
from jax.experimental import pallas as pl
from jax.experimental.pallas import tpu as pltpu
from jax.experimental.pallas import tpu_sc as plsc

N = 16384
E = 262144
B = 64
NPG = N // B
D = 128
HG = 128
HS = 512
T = 512
K_STEPS = 16
ALPHA = 0.1

NC = 2
NS = 16
L = 16

EPT = E // NS
SELF_PT = N // NS
KCH = 128
QROWS = (EPT + SELF_PT) // KCH
QCAP = QROWS * KCH
DH = 32
NP = 128 // DH

_mesh = plsc.VectorSubcoreMesh(core_axis_name="c", subcore_axis_name="s",
                               num_cores=NC, num_subcores=NS)


def _sc_preprocess(edge_ref, qsa_ref, qd_ref, sbs, sbd):
    cid = lax.axis_index("c")
    sid = lax.axis_index("s")
    iot = lax.iota(jnp.int32, L)

    pltpu.sync_copy(edge_ref.at[0, pl.ds(sid * EPT, EPT)], sbs)
    pltpu.sync_copy(edge_ref.at[1, pl.ds(sid * EPT, EPT)], sbd)

    base = cid * NP * N

    def edge_body(i, _):
        s16 = sbs[pl.ds(i * L, L)]
        sbs[pl.ds(i * L, L)] = s16 + base
        return 0

    lax.fori_loop(0, EPT // L, edge_body, 0)
    pltpu.sync_copy(sbs, qsa_ref.at[cid, sid, pl.ds(0, EPT)])
    pltpu.sync_copy(sbd, qd_ref.at[cid, sid, pl.ds(0, EPT)])

    def self_body(i, _):
        node = sid * SELF_PT + i * L + iot
        sbs[pl.ds(i * L, L)] = node + base
        sbd[pl.ds(i * L, L)] = node
        return 0

    lax.fori_loop(0, SELF_PT // L, self_body, 0)
    pltpu.sync_copy(sbs.at[pl.ds(0, SELF_PT)],
                    qsa_ref.at[cid, sid, pl.ds(EPT, SELF_PT)])
    pltpu.sync_copy(sbd.at[pl.ds(0, SELF_PT)],
                    qd_ref.at[cid, sid, pl.ds(EPT, SELF_PT)])


def _sc_iterate(u_in_ref, qsa_ref, qd_ref, br_ref, cc_ref,
                u_out_ref, qsw_v, qd_v, gbuf, tacc, tcc, tbc, z64, acc_ref,
                gsem, ssem):
    cid = lax.axis_index("c")
    sid = lax.axis_index("s")

    pltpu.sync_copy(qsa_ref.at[cid, sid], qsw_v)
    pltpu.sync_copy(qd_ref.at[cid, sid], qd_v)

    def zb(r, _):
        for j in range(DH // L):
            z64[r, pl.ds(j * L, L)] = jnp.zeros((L,), jnp.float32)
        return 0

    lax.fori_loop(0, 64, zb, 0)

    def az(k, _):
        ds = []
        for j in range(4):
            ds.append(pltpu.async_copy(
                z64, acc_ref.at[pl.ds(sid * 1024 + (k * 4 + j) * 64, 64)],
                gsem.at[j]))
        for d in ds:
            d.wait()
        return 0

    lax.fori_loop(0, 4, az, 0)
    plsc.subcore_barrier()

    for p in range(NP):
        if p > 0:
            def shift(r, _):
                for j in range(KCH // L):
                    sl = pl.ds(j * L, L)
                    qsw_v[r, sl] = qsw_v[r, sl] + N
                return 0

            lax.fori_loop(0, QROWS, shift, 0)

        def outer(jj, _):
            base = jj * 8
            gds = []
            for k in range(8):
                gds.append(pltpu.async_copy(
                    u_in_ref.at[qsw_v.at[base + k]], gbuf.at[k], gsem.at[k]))
            sds = []
            for k in range(8):
                gds[k].wait()
                sds.append(pltpu.async_copy(
                    gbuf.at[k], acc_ref.at[qd_v.at[base + k]], ssem.at[k],
                    add=True))
            for k in range(8):
                sds[k].wait()
            return 0

        lax.fori_loop(0, QROWS // 8, outer, 0)
        plsc.subcore_barrier()

        obase = (cid * NP + p) * N

        def blk(b, _):
            lbase = sid * 1024 + b * 128
            d1 = pltpu.async_copy(acc_ref.at[pl.ds(lbase, 128)], tacc,
                                  gsem.at[0])
            d2 = pltpu.async_copy(br_ref.at[pl.ds(lbase, 128)], tbc,
                                  gsem.at[1])
            d3 = pltpu.async_copy(cc_ref.at[pl.ds(obase + lbase, 128)], tcc,
                                  gsem.at[2])
            d1.wait()
            d2.wait()
            d3.wait()
            pltpu.sync_copy(z64, acc_ref.at[pl.ds(lbase, 64)])
            pltpu.sync_copy(z64, acc_ref.at[pl.ds(lbase + 64, 64)])

            def row(i, _):
                for q in range(DH // L):
                    sl = pl.ds(q * L, L)
                    tacc[i, sl] = tacc[i, sl] * tbc[i, sl] + tcc[i, sl]
                return 0

            lax.fori_loop(0, 128, row, 0)
            pltpu.sync_copy(tacc, u_out_ref.at[pl.ds(obase + lbase, 128)])
            return 0

        lax.fori_loop(0, 8, blk, 0)
        plsc.subcore_barrier()


_preprocess_call = functools.partial(
    pl.kernel,
    out_type=[
        jax.ShapeDtypeStruct((NC, NS, QCAP), jnp.int32),
        jax.ShapeDtypeStruct((NC, NS, QCAP), jnp.int32),
    ],
    mesh=_mesh,
    scratch_types=[
        pltpu.VMEM((EPT,), jnp.int32),
        pltpu.VMEM((EPT,), jnp.int32),
    ],
)


_iterate_call = functools.partial(
    pl.kernel,
    out_type=jax.ShapeDtypeStruct((2 * NP * N, DH), jnp.float32),
    mesh=_mesh,
    compiler_params=pltpu.CompilerParams(use_tc_tiling_on_sc=False),
    scratch_types=[
        pltpu.VMEM((QROWS, KCH), jnp.int32),
        pltpu.VMEM((QROWS, KCH), jnp.int32),
        pltpu.VMEM((8, KCH, DH), jnp.float32),
        pltpu.VMEM((128, DH), jnp.float32),
        pltpu.VMEM((128, DH), jnp.float32),
        pltpu.VMEM((128, DH), jnp.float32),
        pltpu.VMEM((64, DH), jnp.float32),
        pltpu.VMEM_SHARED((N, DH), jnp.float32),
        pltpu.SemaphoreType.DMA((8,)),
        pltpu.SemaphoreType.DMA((8,)),
    ],
)


def _appnp_sc(h, edge_index):
    qsa, qd = _preprocess_call(_sc_preprocess)(edge_index)
    qsa = qsa.reshape(NC, NS, QROWS, KCH)
    qd = qd.reshape(NC, NS, QROWS, KCH)
    it = _iterate_call(_sc_iterate)
    deg_rows = it(jnp.ones((2 * NP * N, DH), jnp.float32), qsa, qd,
                  jnp.ones((N, DH), jnp.float32),
                  jnp.zeros((2 * NP * N, DH), jnp.float32))
    deg = deg_rows[:N, 0]
    g = deg ** -0.5
    beta_rows = jnp.broadcast_to(((1.0 - ALPHA) / deg)[:, None], (N, DH))
    u = ((h * g[:, None]).reshape(N, 2, NP, DH)
         .transpose(1, 2, 0, 3).reshape(2 * NP * N, DH))
    cc = ALPHA * u
    for _ in range(K_STEPS):
        u = it(u, qsa, qd, beta_rows, cc)
    hf = (u.reshape(2, NP, N, DH).transpose(2, 0, 1, 3).reshape(N, 2 * HG)
          * jnp.sqrt(deg)[:, None])
    return hf


def _mm_bias_kernel(x_ref, w_ref, b_ref, o_ref, *, act):
    y = jnp.dot(x_ref[...], w_ref[...],
                preferred_element_type=jnp.float32) + b_ref[...]
    if act == "relu":
        y = jnp.maximum(y, 0.0)
    o_ref[...] = y


def _mm_bias(x, w, b, act="none", bm=1024):
    M, K = x.shape
    Nc = w.shape[1]
    if M <= bm:
        bm = M
    grid = (M // bm,)
    return pl.pallas_call(
        functools.partial(_mm_bias_kernel, act=act),
        grid=grid,
        in_specs=[
            pl.BlockSpec((bm, K), lambda i: (i, 0)),
            pl.BlockSpec((K, Nc), lambda i: (0, 0)),
            pl.BlockSpec((1, Nc), lambda i: (0, 0)),
        ],
        out_specs=pl.BlockSpec((bm, Nc), lambda i: (i, 0)),
        out_shape=jax.ShapeDtypeStruct((M, Nc), jnp.float32),
    )(x, w, b.reshape(1, Nc))


def _gru_math(gi, gh, h, H):
    r = jax.nn.sigmoid(gi[:, :H] + gh[:, :H])
    z = jax.nn.sigmoid(gi[:, H:2 * H] + gh[:, H:2 * H])
    n = jnp.tanh(gi[:, 2 * H:] + r * gh[:, 2 * H:])
    return (1.0 - z) * n + z * h


def _seq_scan_kernel(gif_ref, gib_ref, whf_ref, whb_ref, bhf_ref, bhb_ref,
                     o_ref, hf, hb, sumf, sumb, maxf, maxb):
    t = pl.program_id(0)

    @pl.when(t == 0)
    def _():
        hf[...] = jnp.zeros_like(hf)
        hb[...] = jnp.zeros_like(hb)
        sumf[...] = jnp.zeros_like(sumf)
        sumb[...] = jnp.zeros_like(sumb)
        maxf[...] = jnp.full_like(maxf, -jnp.inf)
        maxb[...] = jnp.full_like(maxb, -jnp.inf)

    ghf = jnp.dot(hf[...], whf_ref[...],
                  preferred_element_type=jnp.float32) + bhf_ref[...]
    hfn = _gru_math(gif_ref[0], ghf, hf[...], HS)
    hf[...] = hfn
    sumf[...] += hfn
    maxf[...] = jnp.maximum(maxf[...], hfn)

    ghb = jnp.dot(hb[...], whb_ref[...],
                  preferred_element_type=jnp.float32) + bhb_ref[...]
    hbn = _gru_math(gib_ref[0], ghb, hb[...], HS)
    hb[...] = hbn
    sumb[...] += hbn
    maxb[...] = jnp.maximum(maxb[...], hbn)

    @pl.when(t == T - 1)
    def _():
        o_ref[0] = sumf[...]
        o_ref[1] = sumb[...]
        o_ref[2] = maxf[...]
        o_ref[3] = maxb[...]


def _seq_branch(seq, p):
    x2d = seq.reshape(B * T, D)
    wf = jnp.concatenate([p['sWih_f'].T, p['sWih_b'].T], axis=1)
    bf = jnp.concatenate([p['sbih_f'], p['sbih_b']])
    gi = _mm_bias(x2d, wf, bf)
    gi = gi.reshape(B, T, 2, 3 * HS).transpose(2, 1, 0, 3)
    gif, gib = gi[0], gi[1]
    out = pl.pallas_call(
        _seq_scan_kernel,
        grid=(T,),
        in_specs=[
            pl.BlockSpec((1, B, 3 * HS), lambda t: (t, 0, 0)),
            pl.BlockSpec((1, B, 3 * HS), lambda t: (T - 1 - t, 0, 0)),
            pl.BlockSpec((HS, 3 * HS), lambda t: (0, 0)),
            pl.BlockSpec((HS, 3 * HS), lambda t: (0, 0)),
            pl.BlockSpec((1, 3 * HS), lambda t: (0, 0)),
            pl.BlockSpec((1, 3 * HS), lambda t: (0, 0)),
        ],
        out_specs=pl.BlockSpec((4, B, HS), lambda t: (0, 0, 0)),
        out_shape=jax.ShapeDtypeStruct((4, B, HS), jnp.float32),
        scratch_shapes=[pltpu.VMEM((B, HS), jnp.float32)] * 6,
    )(gif, gib, p['sWhh_f'].T, p['sWhh_b'].T,
      p['sbhh_f'].reshape(1, 3 * HS), p['sbhh_b'].reshape(1, 3 * HS))
    seq1 = jnp.concatenate([out[0], out[1]], axis=1) / T
    seq2 = jnp.concatenate([out[2], out[3]], axis=1)
    return seq1, seq2


def _graph_scan_kernel(gif_ref, gib_ref, whf_ref, whb_ref, bhf_ref, bhb_ref,
                       yf_ref, yb_ref, hf, hb):
    t = pl.program_id(0)

    @pl.when(t == 0)
    def _():
        hf[...] = jnp.zeros_like(hf)
        hb[...] = jnp.zeros_like(hb)

    ghf = jnp.dot(hf[...], whf_ref[...],
                  preferred_element_type=jnp.float32) + bhf_ref[...]
    hfn = _gru_math(gif_ref[0], ghf, hf[...], HG)
    hf[...] = hfn
    yf_ref[0] = hfn

    ghb = jnp.dot(hb[...], whb_ref[...],
                  preferred_element_type=jnp.float32) + bhb_ref[...]
    hbn = _gru_math(gib_ref[0], ghb, hb[...], HG)
    hb[...] = hbn
    yb_ref[0] = hbn


def _graph_branch(features, p):
    wf = jnp.concatenate([p['gWih_f'].T, p['gWih_b'].T], axis=1)
    bf = jnp.concatenate([p['gbih_f'], p['gbih_b']])
    gi = _mm_bias(features, wf, bf)
    gi = gi.reshape(B, NPG, 2, 3 * HG).transpose(2, 1, 0, 3)
    gif, gib = gi[0], gi[1]
    yf, yb = pl.pallas_call(
        _graph_scan_kernel,
        grid=(NPG,),
        in_specs=[
            pl.BlockSpec((1, B, 3 * HG), lambda t: (t, 0, 0)),
            pl.BlockSpec((1, B, 3 * HG), lambda t: (NPG - 1 - t, 0, 0)),
            pl.BlockSpec((HG, 3 * HG), lambda t: (0, 0)),
            pl.BlockSpec((HG, 3 * HG), lambda t: (0, 0)),
            pl.BlockSpec((1, 3 * HG), lambda t: (0, 0)),
            pl.BlockSpec((1, 3 * HG), lambda t: (0, 0)),
        ],
        out_specs=[
            pl.BlockSpec((1, B, HG), lambda t: (t, 0, 0)),
            pl.BlockSpec((1, B, HG), lambda t: (NPG - 1 - t, 0, 0)),
        ],
        out_shape=[
            jax.ShapeDtypeStruct((NPG, B, HG), jnp.float32),
            jax.ShapeDtypeStruct((NPG, B, HG), jnp.float32),
        ],
        scratch_shapes=[pltpu.VMEM((B, HG), jnp.float32)] * 2,
    )(gif, gib, p['gWhh_f'].T, p['gWhh_b'].T,
      p['gbhh_f'].reshape(1, 3 * HG), p['gbhh_b'].reshape(1, 3 * HG))
    st = jnp.concatenate([yf, yb], axis=2).transpose(1, 0, 2)
    return st.reshape(N, 2 * HG)


def _mlp_readout(x, W0, b0, W1, b1, W2, b2):
    h = _mm_bias(x, W0.T, b0, act="relu")
    h = _mm_bias(h, W1.T, b1, act="relu")
    return _mm_bias(h, W2.T, b2)


def kernel(features, edge_index, seq, params):
    p = params
    seq1, seq2 = _seq_branch(seq, p)
    h = _graph_branch(features, p)
    h = _appnp_sc(h, edge_index)
    stg = h.reshape(B, NPG, 2 * HG)
    st1 = jnp.max(stg, axis=1)
    st2 = jnp.mean(stg, axis=1)
    outputs = _mlp_readout(st1 + st2, p['mW0'], p['mb0'], p['mW1'], p['mb1'],
                           p['mW2'], p['mb2'])
    outputs1 = _mlp_readout(seq1 + seq2, p['nW0'], p['nb0'], p['nW1'], p['nb1'],
                            p['nW2'], p['nb2'])
    out = outputs1 + outputs
    return (out, out, out)


# speedup vs baseline: 1.2915x; 1.0688x over previous
import functools

import jax
import jax.numpy as jnp
from jax import lax
from jax.experimental import pallas as pl
from jax.experimental.pallas import tpu as pltpu
from jax.experimental.pallas import tpu_sc as plsc

N = 16384
E = 262144
B = 64
NPG = N // B
D = 128
HG = 128
HS = 512
T = 512
K_STEPS = 16
ALPHA = 0.1

NC = 2
NS = 16
L = 16

EPT = E // NS
SELF_PT = N // NS
KCH = 128
QROWS = (EPT + SELF_PT) // KCH
QCAP = QROWS * KCH
DH = 32
NP = 128 // DH

_mesh = plsc.VectorSubcoreMesh(core_axis_name="c", subcore_axis_name="s",
                               num_cores=NC, num_subcores=NS)


def _sc_preprocess(edge_ref, qsa_ref, qd_ref, sbs, sbd):
    cid = lax.axis_index("c")
    sid = lax.axis_index("s")
    iot = lax.iota(jnp.int32, L)

    pltpu.sync_copy(edge_ref.at[0, pl.ds(sid * EPT, EPT)], sbs)
    pltpu.sync_copy(edge_ref.at[1, pl.ds(sid * EPT, EPT)], sbd)

    base = cid * NP * N

    def edge_body(i, _):
        s16 = sbs[pl.ds(i * L, L)]
        sbs[pl.ds(i * L, L)] = s16 + base
        return 0

    lax.fori_loop(0, EPT // L, edge_body, 0)
    pltpu.sync_copy(sbs, qsa_ref.at[cid, sid, pl.ds(0, EPT)])
    pltpu.sync_copy(sbd, qd_ref.at[cid, sid, pl.ds(0, EPT)])

    def self_body(i, _):
        node = sid * SELF_PT + i * L + iot
        sbs[pl.ds(i * L, L)] = node + base
        sbd[pl.ds(i * L, L)] = node
        return 0

    lax.fori_loop(0, SELF_PT // L, self_body, 0)
    pltpu.sync_copy(sbs.at[pl.ds(0, SELF_PT)],
                    qsa_ref.at[cid, sid, pl.ds(EPT, SELF_PT)])
    pltpu.sync_copy(sbd.at[pl.ds(0, SELF_PT)],
                    qd_ref.at[cid, sid, pl.ds(EPT, SELF_PT)])


def _sc_iterate(u_in_ref, qsa_ref, qd_ref, br_ref, cc_ref,
                u_out_ref, qsw_v, qd_v, gbuf, tacc, tcc, tbc, z64, acc_ref,
                gsem, ssem):
    cid = lax.axis_index("c")
    sid = lax.axis_index("s")

    pltpu.sync_copy(qsa_ref.at[cid, sid], qsw_v)
    pltpu.sync_copy(qd_ref.at[cid, sid], qd_v)

    def zb(r, _):
        for j in range(DH // L):
            z64[r, pl.ds(j * L, L)] = jnp.zeros((L,), jnp.float32)
        return 0

    lax.fori_loop(0, 64, zb, 0)

    def az(k, _):
        ds = []
        for j in range(4):
            ds.append(pltpu.async_copy(
                z64, acc_ref.at[pl.ds(sid * 1024 + (k * 4 + j) * 64, 64)],
                gsem.at[j]))
        for d in ds:
            d.wait()
        return 0

    lax.fori_loop(0, 4, az, 0)
    plsc.subcore_barrier()

    for p in range(NP):
        if p > 0:
            def shift(r, _):
                for j in range(KCH // L):
                    sl = pl.ds(j * L, L)
                    qsw_v[r, sl] = qsw_v[r, sl] + N
                return 0

            lax.fori_loop(0, QROWS, shift, 0)

        def outer(jj, _):
            base = jj * 17
            gds = [None] * 17
            sds = [None] * 17
            for c in range(17):
                b = c % 8
                if c >= 8:
                    sds[c - 8].wait()
                gds[c] = pltpu.async_copy(
                    u_in_ref.at[qsw_v.at[base + c]], gbuf.at[b], gsem.at[b])
                if c >= 4:
                    i = c - 4
                    gds[i].wait()
                    sds[i] = pltpu.async_copy(
                        gbuf.at[i % 8], acc_ref.at[qd_v.at[base + i]],
                        ssem.at[i % 8], add=True)
            for i in range(13, 17):
                gds[i].wait()
                sds[i] = pltpu.async_copy(
                    gbuf.at[i % 8], acc_ref.at[qd_v.at[base + i]],
                    ssem.at[i % 8], add=True)
            for i in range(9, 17):
                sds[i].wait()
            return 0

        lax.fori_loop(0, QROWS // 17, outer, 0)
        plsc.subcore_barrier()

        obase = (cid * NP + p) * N

        def blk(b, _):
            lbase = sid * 1024 + b * 128
            d1 = pltpu.async_copy(acc_ref.at[pl.ds(lbase, 128)], tacc,
                                  gsem.at[0])
            d2 = pltpu.async_copy(br_ref.at[pl.ds(lbase, 128)], tbc,
                                  gsem.at[1])
            d3 = pltpu.async_copy(cc_ref.at[pl.ds(obase + lbase, 128)], tcc,
                                  gsem.at[2])
            d1.wait()
            d2.wait()
            d3.wait()
            pltpu.sync_copy(z64, acc_ref.at[pl.ds(lbase, 64)])
            pltpu.sync_copy(z64, acc_ref.at[pl.ds(lbase + 64, 64)])

            def row(i, _):
                for q in range(DH // L):
                    sl = pl.ds(q * L, L)
                    tacc[i, sl] = tacc[i, sl] * tbc[i, sl] + tcc[i, sl]
                return 0

            lax.fori_loop(0, 128, row, 0)
            pltpu.sync_copy(tacc, u_out_ref.at[pl.ds(obase + lbase, 128)])
            return 0

        lax.fori_loop(0, 8, blk, 0)
        plsc.subcore_barrier()


_preprocess_call = functools.partial(
    pl.kernel,
    out_type=[
        jax.ShapeDtypeStruct((NC, NS, QCAP), jnp.int32),
        jax.ShapeDtypeStruct((NC, NS, QCAP), jnp.int32),
    ],
    mesh=_mesh,
    scratch_types=[
        pltpu.VMEM((EPT,), jnp.int32),
        pltpu.VMEM((EPT,), jnp.int32),
    ],
)


_iterate_call = functools.partial(
    pl.kernel,
    out_type=jax.ShapeDtypeStruct((2 * NP * N, DH), jnp.float32),
    mesh=_mesh,
    compiler_params=pltpu.CompilerParams(use_tc_tiling_on_sc=False),
    scratch_types=[
        pltpu.VMEM((QROWS, KCH), jnp.int32),
        pltpu.VMEM((QROWS, KCH), jnp.int32),
        pltpu.VMEM((8, KCH, DH), jnp.float32),
        pltpu.VMEM((128, DH), jnp.float32),
        pltpu.VMEM((128, DH), jnp.float32),
        pltpu.VMEM((128, DH), jnp.float32),
        pltpu.VMEM((64, DH), jnp.float32),
        pltpu.VMEM_SHARED((N, DH), jnp.float32),
        pltpu.SemaphoreType.DMA((8,)),
        pltpu.SemaphoreType.DMA((8,)),
    ],
)


def _appnp_sc(h, edge_index):
    qsa, qd = _preprocess_call(_sc_preprocess)(edge_index)
    qsa = qsa.reshape(NC, NS, QROWS, KCH)
    qd = qd.reshape(NC, NS, QROWS, KCH)
    it = _iterate_call(_sc_iterate)
    deg_rows = it(jnp.ones((2 * NP * N, DH), jnp.float32), qsa, qd,
                  jnp.ones((N, DH), jnp.float32),
                  jnp.zeros((2 * NP * N, DH), jnp.float32))
    deg = deg_rows[:N, 0]
    g = deg ** -0.5
    beta_rows = jnp.broadcast_to(((1.0 - ALPHA) / deg)[:, None], (N, DH))
    u = ((h * g[:, None]).reshape(N, 2, NP, DH)
         .transpose(1, 2, 0, 3).reshape(2 * NP * N, DH))
    cc = ALPHA * u
    for _ in range(K_STEPS):
        u = it(u, qsa, qd, beta_rows, cc)
    hf = (u.reshape(2, NP, N, DH).transpose(2, 0, 1, 3).reshape(N, 2 * HG)
          * jnp.sqrt(deg)[:, None])
    return hf


def _mm_bias_kernel(x_ref, w_ref, b_ref, o_ref, *, act):
    y = jnp.dot(x_ref[...], w_ref[...],
                preferred_element_type=jnp.float32) + b_ref[...]
    if act == "relu":
        y = jnp.maximum(y, 0.0)
    o_ref[...] = y


def _mm_bias(x, w, b, act="none", bm=1024):
    M, K = x.shape
    Nc = w.shape[1]
    if M <= bm:
        bm = M
    grid = (M // bm,)
    return pl.pallas_call(
        functools.partial(_mm_bias_kernel, act=act),
        grid=grid,
        in_specs=[
            pl.BlockSpec((bm, K), lambda i: (i, 0)),
            pl.BlockSpec((K, Nc), lambda i: (0, 0)),
            pl.BlockSpec((1, Nc), lambda i: (0, 0)),
        ],
        out_specs=pl.BlockSpec((bm, Nc), lambda i: (i, 0)),
        out_shape=jax.ShapeDtypeStruct((M, Nc), jnp.float32),
    )(x, w, b.reshape(1, Nc))


def _gru_math(gi, gh, h, H):
    r = jax.nn.sigmoid(gi[:, :H] + gh[:, :H])
    z = jax.nn.sigmoid(gi[:, H:2 * H] + gh[:, H:2 * H])
    n = jnp.tanh(gi[:, 2 * H:] + r * gh[:, 2 * H:])
    return (1.0 - z) * n + z * h


def _seq_scan_kernel(gif_ref, gib_ref, whf_ref, whb_ref, bhf_ref, bhb_ref,
                     o_ref, hf, hb, sumf, sumb, maxf, maxb):
    t = pl.program_id(0)

    @pl.when(t == 0)
    def _():
        hf[...] = jnp.zeros_like(hf)
        hb[...] = jnp.zeros_like(hb)
        sumf[...] = jnp.zeros_like(sumf)
        sumb[...] = jnp.zeros_like(sumb)
        maxf[...] = jnp.full_like(maxf, -jnp.inf)
        maxb[...] = jnp.full_like(maxb, -jnp.inf)

    ghf = jnp.dot(hf[...], whf_ref[...],
                  preferred_element_type=jnp.float32) + bhf_ref[...]
    hfn = _gru_math(gif_ref[0], ghf, hf[...], HS)
    hf[...] = hfn
    sumf[...] += hfn
    maxf[...] = jnp.maximum(maxf[...], hfn)

    ghb = jnp.dot(hb[...], whb_ref[...],
                  preferred_element_type=jnp.float32) + bhb_ref[...]
    hbn = _gru_math(gib_ref[0], ghb, hb[...], HS)
    hb[...] = hbn
    sumb[...] += hbn
    maxb[...] = jnp.maximum(maxb[...], hbn)

    @pl.when(t == T - 1)
    def _():
        o_ref[0] = sumf[...]
        o_ref[1] = sumb[...]
        o_ref[2] = maxf[...]
        o_ref[3] = maxb[...]


def _seq_branch(seq, p):
    x2d = seq.reshape(B * T, D)
    wf = jnp.concatenate([p['sWih_f'].T, p['sWih_b'].T], axis=1)
    bf = jnp.concatenate([p['sbih_f'], p['sbih_b']])
    gi = _mm_bias(x2d, wf, bf)
    gi = gi.reshape(B, T, 2, 3 * HS).transpose(2, 1, 0, 3)
    gif, gib = gi[0], gi[1]
    out = pl.pallas_call(
        _seq_scan_kernel,
        grid=(T,),
        in_specs=[
            pl.BlockSpec((1, B, 3 * HS), lambda t: (t, 0, 0)),
            pl.BlockSpec((1, B, 3 * HS), lambda t: (T - 1 - t, 0, 0)),
            pl.BlockSpec((HS, 3 * HS), lambda t: (0, 0)),
            pl.BlockSpec((HS, 3 * HS), lambda t: (0, 0)),
            pl.BlockSpec((1, 3 * HS), lambda t: (0, 0)),
            pl.BlockSpec((1, 3 * HS), lambda t: (0, 0)),
        ],
        out_specs=pl.BlockSpec((4, B, HS), lambda t: (0, 0, 0)),
        out_shape=jax.ShapeDtypeStruct((4, B, HS), jnp.float32),
        scratch_shapes=[pltpu.VMEM((B, HS), jnp.float32)] * 6,
    )(gif, gib, p['sWhh_f'].T, p['sWhh_b'].T,
      p['sbhh_f'].reshape(1, 3 * HS), p['sbhh_b'].reshape(1, 3 * HS))
    seq1 = jnp.concatenate([out[0], out[1]], axis=1) / T
    seq2 = jnp.concatenate([out[2], out[3]], axis=1)
    return seq1, seq2


def _graph_scan_kernel(gif_ref, gib_ref, whf_ref, whb_ref, bhf_ref, bhb_ref,
                       yf_ref, yb_ref, hf, hb):
    t = pl.program_id(0)

    @pl.when(t == 0)
    def _():
        hf[...] = jnp.zeros_like(hf)
        hb[...] = jnp.zeros_like(hb)

    ghf = jnp.dot(hf[...], whf_ref[...],
                  preferred_element_type=jnp.float32) + bhf_ref[...]
    hfn = _gru_math(gif_ref[0], ghf, hf[...], HG)
    hf[...] = hfn
    yf_ref[0] = hfn

    ghb = jnp.dot(hb[...], whb_ref[...],
                  preferred_element_type=jnp.float32) + bhb_ref[...]
    hbn = _gru_math(gib_ref[0], ghb, hb[...], HG)
    hb[...] = hbn
    yb_ref[0] = hbn


def _graph_branch(features, p):
    wf = jnp.concatenate([p['gWih_f'].T, p['gWih_b'].T], axis=1)
    bf = jnp.concatenate([p['gbih_f'], p['gbih_b']])
    gi = _mm_bias(features, wf, bf)
    gi = gi.reshape(B, NPG, 2, 3 * HG).transpose(2, 1, 0, 3)
    gif, gib = gi[0], gi[1]
    yf, yb = pl.pallas_call(
        _graph_scan_kernel,
        grid=(NPG,),
        in_specs=[
            pl.BlockSpec((1, B, 3 * HG), lambda t: (t, 0, 0)),
            pl.BlockSpec((1, B, 3 * HG), lambda t: (NPG - 1 - t, 0, 0)),
            pl.BlockSpec((HG, 3 * HG), lambda t: (0, 0)),
            pl.BlockSpec((HG, 3 * HG), lambda t: (0, 0)),
            pl.BlockSpec((1, 3 * HG), lambda t: (0, 0)),
            pl.BlockSpec((1, 3 * HG), lambda t: (0, 0)),
        ],
        out_specs=[
            pl.BlockSpec((1, B, HG), lambda t: (t, 0, 0)),
            pl.BlockSpec((1, B, HG), lambda t: (NPG - 1 - t, 0, 0)),
        ],
        out_shape=[
            jax.ShapeDtypeStruct((NPG, B, HG), jnp.float32),
            jax.ShapeDtypeStruct((NPG, B, HG), jnp.float32),
        ],
        scratch_shapes=[pltpu.VMEM((B, HG), jnp.float32)] * 2,
    )(gif, gib, p['gWhh_f'].T, p['gWhh_b'].T,
      p['gbhh_f'].reshape(1, 3 * HG), p['gbhh_b'].reshape(1, 3 * HG))
    st = jnp.concatenate([yf, yb], axis=2).transpose(1, 0, 2)
    return st.reshape(N, 2 * HG)


def _mlp_readout(x, W0, b0, W1, b1, W2, b2):
    h = _mm_bias(x, W0.T, b0, act="relu")
    h = _mm_bias(h, W1.T, b1, act="relu")
    return _mm_bias(h, W2.T, b2)


def kernel(features, edge_index, seq, params):
    p = params
    seq1, seq2 = _seq_branch(seq, p)
    h = _graph_branch(features, p)
    h = _appnp_sc(h, edge_index)
    stg = h.reshape(B, NPG, 2 * HG)
    st1 = jnp.max(stg, axis=1)
    st2 = jnp.mean(stg, axis=1)
    outputs = _mlp_readout(st1 + st2, p['mW0'], p['mb0'], p['mW1'], p['mb1'],
                           p['mW2'], p['mb2'])
    outputs1 = _mlp_readout(seq1 + seq2, p['nW0'], p['nb0'], p['nW1'], p['nb1'],
                            p['nW2'], p['nb2'])
    out = outputs1 + outputs
    return (out, out, out)


# speedup vs baseline: 1.3175x; 1.0201x over previous
import functools

import jax
import jax.numpy as jnp
from jax import lax
from jax.experimental import pallas as pl
from jax.experimental.pallas import tpu as pltpu
from jax.experimental.pallas import tpu_sc as plsc

N = 16384
E = 262144
B = 64
NPG = N // B
D = 128
HG = 128
HS = 512
T = 512
K_STEPS = 16
ALPHA = 0.1

NC = 2
NS = 16
L = 16

EPT = E // NS
SELF_PT = N // NS
KCH = 128
QROWS = (EPT + SELF_PT) // KCH
QCAP = QROWS * KCH
DH = 32
NP = 128 // DH

_mesh = plsc.VectorSubcoreMesh(core_axis_name="c", subcore_axis_name="s",
                               num_cores=NC, num_subcores=NS)


def _sc_preprocess(edge_ref, qsa_ref, qd_ref, sbs, sbd):
    cid = lax.axis_index("c")
    sid = lax.axis_index("s")
    iot = lax.iota(jnp.int32, L)

    pltpu.sync_copy(edge_ref.at[0, pl.ds(sid * EPT, EPT)], sbs)
    pltpu.sync_copy(edge_ref.at[1, pl.ds(sid * EPT, EPT)], sbd)

    base = cid * NP * N

    def edge_body(i, _):
        s16 = sbs[pl.ds(i * L, L)]
        sbs[pl.ds(i * L, L)] = s16 + base
        return 0

    lax.fori_loop(0, EPT // L, edge_body, 0)
    pltpu.sync_copy(sbs, qsa_ref.at[cid, sid, pl.ds(0, EPT)])
    pltpu.sync_copy(sbd, qd_ref.at[cid, sid, pl.ds(0, EPT)])

    def self_body(i, _):
        node = sid * SELF_PT + i * L + iot
        sbs[pl.ds(i * L, L)] = node + base
        sbd[pl.ds(i * L, L)] = node
        return 0

    lax.fori_loop(0, SELF_PT // L, self_body, 0)
    pltpu.sync_copy(sbs.at[pl.ds(0, SELF_PT)],
                    qsa_ref.at[cid, sid, pl.ds(EPT, SELF_PT)])
    pltpu.sync_copy(sbd.at[pl.ds(0, SELF_PT)],
                    qd_ref.at[cid, sid, pl.ds(EPT, SELF_PT)])


def _sc_iterate(u_in_ref, qsa_ref, qd_ref, br_ref, cc_ref,
                u_out_ref, qsw_v, qd_v, gbuf, tacc, tcc, tbc, z64, acc_ref,
                gsem, ssem):
    cid = lax.axis_index("c")
    sid = lax.axis_index("s")

    pltpu.sync_copy(qsa_ref.at[cid, sid], qsw_v)
    pltpu.sync_copy(qd_ref.at[cid, sid], qd_v)

    def zb(r, _):
        for j in range(DH // L):
            z64[r, pl.ds(j * L, L)] = jnp.zeros((L,), jnp.float32)
        return 0

    lax.fori_loop(0, 64, zb, 0)

    def az(k, _):
        ds = []
        for j in range(4):
            ds.append(pltpu.async_copy(
                z64, acc_ref.at[pl.ds(sid * 1024 + (k * 4 + j) * 64, 64)],
                gsem.at[j]))
        for d in ds:
            d.wait()
        return 0

    lax.fori_loop(0, 4, az, 0)
    plsc.subcore_barrier()

    for p in range(NP):
        if p > 0:
            def shift(r, _):
                for j in range(KCH // L):
                    sl = pl.ds(j * L, L)
                    qsw_v[r, sl] = qsw_v[r, sl] + N
                return 0

            lax.fori_loop(0, QROWS, shift, 0)

        def outer(jj, _):
            base = jj * 34
            gds = [None] * 34
            sds = [None] * 34
            for c in range(34):
                b = c % 8
                if c >= 8:
                    sds[c - 8].wait()
                gds[c] = pltpu.async_copy(
                    u_in_ref.at[qsw_v.at[base + c]], gbuf.at[b], gsem.at[b])
                if c >= 4:
                    i = c - 4
                    gds[i].wait()
                    sds[i] = pltpu.async_copy(
                        gbuf.at[i % 8], acc_ref.at[qd_v.at[base + i]],
                        ssem.at[i % 8], add=True)
            for i in range(30, 34):
                gds[i].wait()
                sds[i] = pltpu.async_copy(
                    gbuf.at[i % 8], acc_ref.at[qd_v.at[base + i]],
                    ssem.at[i % 8], add=True)
            for i in range(26, 34):
                sds[i].wait()
            return 0

        lax.fori_loop(0, QROWS // 34, outer, 0)
        plsc.subcore_barrier()

        obase = (cid * NP + p) * N

        def blk(b, _):
            lbase = sid * 1024 + b * 128
            d1 = pltpu.async_copy(acc_ref.at[pl.ds(lbase, 128)], tacc,
                                  gsem.at[0])
            d2 = pltpu.async_copy(br_ref.at[pl.ds(lbase, 128)], tbc,
                                  gsem.at[1])
            d3 = pltpu.async_copy(cc_ref.at[pl.ds(obase + lbase, 128)], tcc,
                                  gsem.at[2])
            d1.wait()
            d2.wait()
            d3.wait()
            pltpu.sync_copy(z64, acc_ref.at[pl.ds(lbase, 64)])
            pltpu.sync_copy(z64, acc_ref.at[pl.ds(lbase + 64, 64)])

            def row(i, _):
                for q in range(DH // L):
                    sl = pl.ds(q * L, L)
                    tacc[i, sl] = tacc[i, sl] * tbc[i, sl] + tcc[i, sl]
                return 0

            lax.fori_loop(0, 128, row, 0)
            pltpu.sync_copy(tacc, u_out_ref.at[pl.ds(obase + lbase, 128)])
            return 0

        lax.fori_loop(0, 8, blk, 0)
        plsc.subcore_barrier()


_preprocess_call = functools.partial(
    pl.kernel,
    out_type=[
        jax.ShapeDtypeStruct((NC, NS, QCAP), jnp.int32),
        jax.ShapeDtypeStruct((NC, NS, QCAP), jnp.int32),
    ],
    mesh=_mesh,
    scratch_types=[
        pltpu.VMEM((EPT,), jnp.int32),
        pltpu.VMEM((EPT,), jnp.int32),
    ],
)


_iterate_call = functools.partial(
    pl.kernel,
    out_type=jax.ShapeDtypeStruct((2 * NP * N, DH), jnp.float32),
    mesh=_mesh,
    compiler_params=pltpu.CompilerParams(use_tc_tiling_on_sc=False),
    scratch_types=[
        pltpu.VMEM((QROWS, KCH), jnp.int32),
        pltpu.VMEM((QROWS, KCH), jnp.int32),
        pltpu.VMEM((8, KCH, DH), jnp.float32),
        pltpu.VMEM((128, DH), jnp.float32),
        pltpu.VMEM((128, DH), jnp.float32),
        pltpu.VMEM((128, DH), jnp.float32),
        pltpu.VMEM((64, DH), jnp.float32),
        pltpu.VMEM_SHARED((N, DH), jnp.float32),
        pltpu.SemaphoreType.DMA((8,)),
        pltpu.SemaphoreType.DMA((8,)),
    ],
)


def _appnp_sc(h, edge_index):
    qsa, qd = _preprocess_call(_sc_preprocess)(edge_index)
    qsa = qsa.reshape(NC, NS, QROWS, KCH)
    qd = qd.reshape(NC, NS, QROWS, KCH)
    it = _iterate_call(_sc_iterate)
    deg_rows = it(jnp.ones((2 * NP * N, DH), jnp.float32), qsa, qd,
                  jnp.ones((N, DH), jnp.float32),
                  jnp.zeros((2 * NP * N, DH), jnp.float32))
    deg = deg_rows[:N, 0]
    g = deg ** -0.5
    beta_rows = jnp.broadcast_to(((1.0 - ALPHA) / deg)[:, None], (N, DH))
    u = ((h * g[:, None]).reshape(N, 2, NP, DH)
         .transpose(1, 2, 0, 3).reshape(2 * NP * N, DH))
    cc = ALPHA * u
    for _ in range(K_STEPS):
        u = it(u, qsa, qd, beta_rows, cc)
    hf = (u.reshape(2, NP, N, DH).transpose(2, 0, 1, 3).reshape(N, 2 * HG)
          * jnp.sqrt(deg)[:, None])
    return hf


def _mm_bias_kernel(x_ref, w_ref, b_ref, o_ref, *, act):
    y = jnp.dot(x_ref[...], w_ref[...],
                preferred_element_type=jnp.float32) + b_ref[...]
    if act == "relu":
        y = jnp.maximum(y, 0.0)
    o_ref[...] = y


def _mm_bias(x, w, b, act="none", bm=1024):
    M, K = x.shape
    Nc = w.shape[1]
    if M <= bm:
        bm = M
    grid = (M // bm,)
    return pl.pallas_call(
        functools.partial(_mm_bias_kernel, act=act),
        grid=grid,
        in_specs=[
            pl.BlockSpec((bm, K), lambda i: (i, 0)),
            pl.BlockSpec((K, Nc), lambda i: (0, 0)),
            pl.BlockSpec((1, Nc), lambda i: (0, 0)),
        ],
        out_specs=pl.BlockSpec((bm, Nc), lambda i: (i, 0)),
        out_shape=jax.ShapeDtypeStruct((M, Nc), jnp.float32),
    )(x, w, b.reshape(1, Nc))


def _gru_math(gi, gh, h, H):
    r = jax.nn.sigmoid(gi[:, :H] + gh[:, :H])
    z = jax.nn.sigmoid(gi[:, H:2 * H] + gh[:, H:2 * H])
    n = jnp.tanh(gi[:, 2 * H:] + r * gh[:, 2 * H:])
    return (1.0 - z) * n + z * h


def _seq_scan_kernel(gif_ref, gib_ref, whf_ref, whb_ref, bhf_ref, bhb_ref,
                     o_ref, hf, hb, sumf, sumb, maxf, maxb):
    t = pl.program_id(0)

    @pl.when(t == 0)
    def _():
        hf[...] = jnp.zeros_like(hf)
        hb[...] = jnp.zeros_like(hb)
        sumf[...] = jnp.zeros_like(sumf)
        sumb[...] = jnp.zeros_like(sumb)
        maxf[...] = jnp.full_like(maxf, -jnp.inf)
        maxb[...] = jnp.full_like(maxb, -jnp.inf)

    ghf = jnp.dot(hf[...], whf_ref[...],
                  preferred_element_type=jnp.float32) + bhf_ref[...]
    hfn = _gru_math(gif_ref[0], ghf, hf[...], HS)
    hf[...] = hfn
    sumf[...] += hfn
    maxf[...] = jnp.maximum(maxf[...], hfn)

    ghb = jnp.dot(hb[...], whb_ref[...],
                  preferred_element_type=jnp.float32) + bhb_ref[...]
    hbn = _gru_math(gib_ref[0], ghb, hb[...], HS)
    hb[...] = hbn
    sumb[...] += hbn
    maxb[...] = jnp.maximum(maxb[...], hbn)

    @pl.when(t == T - 1)
    def _():
        o_ref[0] = sumf[...]
        o_ref[1] = sumb[...]
        o_ref[2] = maxf[...]
        o_ref[3] = maxb[...]


def _seq_branch(seq, p):
    x2d = seq.reshape(B * T, D)
    wf = jnp.concatenate([p['sWih_f'].T, p['sWih_b'].T], axis=1)
    bf = jnp.concatenate([p['sbih_f'], p['sbih_b']])
    gi = _mm_bias(x2d, wf, bf)
    gi = gi.reshape(B, T, 2, 3 * HS).transpose(2, 1, 0, 3)
    gif, gib = gi[0], gi[1]
    out = pl.pallas_call(
        _seq_scan_kernel,
        grid=(T,),
        in_specs=[
            pl.BlockSpec((1, B, 3 * HS), lambda t: (t, 0, 0)),
            pl.BlockSpec((1, B, 3 * HS), lambda t: (T - 1 - t, 0, 0)),
            pl.BlockSpec((HS, 3 * HS), lambda t: (0, 0)),
            pl.BlockSpec((HS, 3 * HS), lambda t: (0, 0)),
            pl.BlockSpec((1, 3 * HS), lambda t: (0, 0)),
            pl.BlockSpec((1, 3 * HS), lambda t: (0, 0)),
        ],
        out_specs=pl.BlockSpec((4, B, HS), lambda t: (0, 0, 0)),
        out_shape=jax.ShapeDtypeStruct((4, B, HS), jnp.float32),
        scratch_shapes=[pltpu.VMEM((B, HS), jnp.float32)] * 6,
    )(gif, gib, p['sWhh_f'].T, p['sWhh_b'].T,
      p['sbhh_f'].reshape(1, 3 * HS), p['sbhh_b'].reshape(1, 3 * HS))
    seq1 = jnp.concatenate([out[0], out[1]], axis=1) / T
    seq2 = jnp.concatenate([out[2], out[3]], axis=1)
    return seq1, seq2


def _graph_scan_kernel(gif_ref, gib_ref, whf_ref, whb_ref, bhf_ref, bhb_ref,
                       yf_ref, yb_ref, hf, hb):
    t = pl.program_id(0)

    @pl.when(t == 0)
    def _():
        hf[...] = jnp.zeros_like(hf)
        hb[...] = jnp.zeros_like(hb)

    ghf = jnp.dot(hf[...], whf_ref[...],
                  preferred_element_type=jnp.float32) + bhf_ref[...]
    hfn = _gru_math(gif_ref[0], ghf, hf[...], HG)
    hf[...] = hfn
    yf_ref[0] = hfn

    ghb = jnp.dot(hb[...], whb_ref[...],
                  preferred_element_type=jnp.float32) + bhb_ref[...]
    hbn = _gru_math(gib_ref[0], ghb, hb[...], HG)
    hb[...] = hbn
    yb_ref[0] = hbn


def _graph_branch(features, p):
    wf = jnp.concatenate([p['gWih_f'].T, p['gWih_b'].T], axis=1)
    bf = jnp.concatenate([p['gbih_f'], p['gbih_b']])
    gi = _mm_bias(features, wf, bf)
    gi = gi.reshape(B, NPG, 2, 3 * HG).transpose(2, 1, 0, 3)
    gif, gib = gi[0], gi[1]
    yf, yb = pl.pallas_call(
        _graph_scan_kernel,
        grid=(NPG,),
        in_specs=[
            pl.BlockSpec((1, B, 3 * HG), lambda t: (t, 0, 0)),
            pl.BlockSpec((1, B, 3 * HG), lambda t: (NPG - 1 - t, 0, 0)),
            pl.BlockSpec((HG, 3 * HG), lambda t: (0, 0)),
            pl.BlockSpec((HG, 3 * HG), lambda t: (0, 0)),
            pl.BlockSpec((1, 3 * HG), lambda t: (0, 0)),
            pl.BlockSpec((1, 3 * HG), lambda t: (0, 0)),
        ],
        out_specs=[
            pl.BlockSpec((1, B, HG), lambda t: (t, 0, 0)),
            pl.BlockSpec((1, B, HG), lambda t: (NPG - 1 - t, 0, 0)),
        ],
        out_shape=[
            jax.ShapeDtypeStruct((NPG, B, HG), jnp.float32),
            jax.ShapeDtypeStruct((NPG, B, HG), jnp.float32),
        ],
        scratch_shapes=[pltpu.VMEM((B, HG), jnp.float32)] * 2,
    )(gif, gib, p['gWhh_f'].T, p['gWhh_b'].T,
      p['gbhh_f'].reshape(1, 3 * HG), p['gbhh_b'].reshape(1, 3 * HG))
    st = jnp.concatenate([yf, yb], axis=2).transpose(1, 0, 2)
    return st.reshape(N, 2 * HG)


def _mlp_readout(x, W0, b0, W1, b1, W2, b2):
    h = _mm_bias(x, W0.T, b0, act="relu")
    h = _mm_bias(h, W1.T, b1, act="relu")
    return _mm_bias(h, W2.T, b2)


def kernel(features, edge_index, seq, params):
    p = params
    seq1, seq2 = _seq_branch(seq, p)
    h = _graph_branch(features, p)
    h = _appnp_sc(h, edge_index)
    stg = h.reshape(B, NPG, 2 * HG)
    st1 = jnp.max(stg, axis=1)
    st2 = jnp.mean(stg, axis=1)
    outputs = _mlp_readout(st1 + st2, p['mW0'], p['mb0'], p['mW1'], p['mb1'],
                           p['mW2'], p['mb2'])
    outputs1 = _mlp_readout(seq1 + seq2, p['nW0'], p['nb0'], p['nW1'], p['nb1'],
                            p['nW2'], p['nb2'])
    out = outputs1 + outputs
    return (out, out, out)


# speedup vs baseline: 1.3497x; 1.0244x over previous
import functools

import jax
import jax.numpy as jnp
from jax import lax
from jax.experimental import pallas as pl
from jax.experimental.pallas import tpu as pltpu
from jax.experimental.pallas import tpu_sc as plsc

N = 16384
E = 262144
B = 64
NPG = N // B
D = 128
HG = 128
HS = 512
T = 512
K_STEPS = 16
ALPHA = 0.1

NC = 2
NS = 16
L = 16

EPT = E // NS
SELF_PT = N // NS
KCH = 128
QROWS = (EPT + SELF_PT) // KCH
QCAP = QROWS * KCH
DH = 32
NP = 128 // DH

_mesh = plsc.VectorSubcoreMesh(core_axis_name="c", subcore_axis_name="s",
                               num_cores=NC, num_subcores=NS)


def _sc_preprocess(edge_ref, qsa_ref, qd_ref, sbs, sbd):
    cid = lax.axis_index("c")
    sid = lax.axis_index("s")
    iot = lax.iota(jnp.int32, L)

    pltpu.sync_copy(edge_ref.at[0, pl.ds(sid * EPT, EPT)], sbs)
    pltpu.sync_copy(edge_ref.at[1, pl.ds(sid * EPT, EPT)], sbd)

    base = cid * NP * N

    def edge_body(i, _):
        s16 = sbs[pl.ds(i * L, L)]
        sbs[pl.ds(i * L, L)] = s16 + base
        return 0

    lax.fori_loop(0, EPT // L, edge_body, 0)
    pltpu.sync_copy(sbs, qsa_ref.at[cid, sid, pl.ds(0, EPT)])
    pltpu.sync_copy(sbd, qd_ref.at[cid, sid, pl.ds(0, EPT)])

    def self_body(i, _):
        node = sid * SELF_PT + i * L + iot
        sbs[pl.ds(i * L, L)] = node + base
        sbd[pl.ds(i * L, L)] = node
        return 0

    lax.fori_loop(0, SELF_PT // L, self_body, 0)
    pltpu.sync_copy(sbs.at[pl.ds(0, SELF_PT)],
                    qsa_ref.at[cid, sid, pl.ds(EPT, SELF_PT)])
    pltpu.sync_copy(sbd.at[pl.ds(0, SELF_PT)],
                    qd_ref.at[cid, sid, pl.ds(EPT, SELF_PT)])


def _sc_iterate(u_in_ref, qsa_ref, qd_ref, br_ref, cc_ref,
                u_out_ref, qsw_v, qd_v, gbuf, tacc, tcc, tbc, z64, acc_ref,
                gsem, ssem):
    cid = lax.axis_index("c")
    sid = lax.axis_index("s")

    pltpu.sync_copy(qsa_ref.at[cid, sid], qsw_v)
    pltpu.sync_copy(qd_ref.at[cid, sid], qd_v)

    def zb(r, _):
        for j in range(DH // L):
            z64[r, pl.ds(j * L, L)] = jnp.zeros((L,), jnp.float32)
        return 0

    lax.fori_loop(0, 64, zb, 0)

    def az(k, _):
        ds = []
        for j in range(4):
            ds.append(pltpu.async_copy(
                z64, acc_ref.at[pl.ds(sid * 1024 + (k * 4 + j) * 64, 64)],
                gsem.at[j]))
        for d in ds:
            d.wait()
        return 0

    lax.fori_loop(0, 4, az, 0)
    plsc.subcore_barrier()

    for p in range(NP):
        if p > 0:
            def shift(r, _):
                for j in range(KCH // L):
                    sl = pl.ds(j * L, L)
                    qsw_v[r, sl] = qsw_v[r, sl] + N
                return 0

            lax.fori_loop(0, QROWS, shift, 0)

        def outer(jj, _):
            base = jj * 68
            gds = [None] * 68
            sds = [None] * 68
            for c in range(68):
                b = c % 8
                if c >= 8:
                    sds[c - 8].wait()
                gds[c] = pltpu.async_copy(
                    u_in_ref.at[qsw_v.at[base + c]], gbuf.at[b], gsem.at[b])
                if c >= 4:
                    i = c - 4
                    gds[i].wait()
                    sds[i] = pltpu.async_copy(
                        gbuf.at[i % 8], acc_ref.at[qd_v.at[base + i]],
                        ssem.at[i % 8], add=True)
            for i in range(64, 68):
                gds[i].wait()
                sds[i] = pltpu.async_copy(
                    gbuf.at[i % 8], acc_ref.at[qd_v.at[base + i]],
                    ssem.at[i % 8], add=True)
            for i in range(60, 68):
                sds[i].wait()
            return 0

        lax.fori_loop(0, QROWS // 68, outer, 0)
        plsc.subcore_barrier()

        obase = (cid * NP + p) * N

        def blk(b, _):
            lbase = sid * 1024 + b * 128
            d1 = pltpu.async_copy(acc_ref.at[pl.ds(lbase, 128)], tacc,
                                  gsem.at[0])
            d2 = pltpu.async_copy(br_ref.at[pl.ds(lbase, 128)], tbc,
                                  gsem.at[1])
            d3 = pltpu.async_copy(cc_ref.at[pl.ds(obase + lbase, 128)], tcc,
                                  gsem.at[2])
            d1.wait()
            d2.wait()
            d3.wait()
            pltpu.sync_copy(z64, acc_ref.at[pl.ds(lbase, 64)])
            pltpu.sync_copy(z64, acc_ref.at[pl.ds(lbase + 64, 64)])

            def row(i, _):
                for q in range(DH // L):
                    sl = pl.ds(q * L, L)
                    tacc[i, sl] = tacc[i, sl] * tbc[i, sl] + tcc[i, sl]
                return 0

            lax.fori_loop(0, 128, row, 0)
            pltpu.sync_copy(tacc, u_out_ref.at[pl.ds(obase + lbase, 128)])
            return 0

        lax.fori_loop(0, 8, blk, 0)
        plsc.subcore_barrier()


_preprocess_call = functools.partial(
    pl.kernel,
    out_type=[
        jax.ShapeDtypeStruct((NC, NS, QCAP), jnp.int32),
        jax.ShapeDtypeStruct((NC, NS, QCAP), jnp.int32),
    ],
    mesh=_mesh,
    scratch_types=[
        pltpu.VMEM((EPT,), jnp.int32),
        pltpu.VMEM((EPT,), jnp.int32),
    ],
)


_iterate_call = functools.partial(
    pl.kernel,
    out_type=jax.ShapeDtypeStruct((2 * NP * N, DH), jnp.float32),
    mesh=_mesh,
    compiler_params=pltpu.CompilerParams(use_tc_tiling_on_sc=False),
    scratch_types=[
        pltpu.VMEM((QROWS, KCH), jnp.int32),
        pltpu.VMEM((QROWS, KCH), jnp.int32),
        pltpu.VMEM((8, KCH, DH), jnp.float32),
        pltpu.VMEM((128, DH), jnp.float32),
        pltpu.VMEM((128, DH), jnp.float32),
        pltpu.VMEM((128, DH), jnp.float32),
        pltpu.VMEM((64, DH), jnp.float32),
        pltpu.VMEM_SHARED((N, DH), jnp.float32),
        pltpu.SemaphoreType.DMA((8,)),
        pltpu.SemaphoreType.DMA((8,)),
    ],
)


def _appnp_sc(h, edge_index):
    qsa, qd = _preprocess_call(_sc_preprocess)(edge_index)
    qsa = qsa.reshape(NC, NS, QROWS, KCH)
    qd = qd.reshape(NC, NS, QROWS, KCH)
    it = _iterate_call(_sc_iterate)
    deg_rows = it(jnp.ones((2 * NP * N, DH), jnp.float32), qsa, qd,
                  jnp.ones((N, DH), jnp.float32),
                  jnp.zeros((2 * NP * N, DH), jnp.float32))
    deg = deg_rows[:N, 0]
    g = deg ** -0.5
    beta_rows = jnp.broadcast_to(((1.0 - ALPHA) / deg)[:, None], (N, DH))
    u = ((h * g[:, None]).reshape(N, 2, NP, DH)
         .transpose(1, 2, 0, 3).reshape(2 * NP * N, DH))
    cc = ALPHA * u
    for _ in range(K_STEPS):
        u = it(u, qsa, qd, beta_rows, cc)
    hf = (u.reshape(2, NP, N, DH).transpose(2, 0, 1, 3).reshape(N, 2 * HG)
          * jnp.sqrt(deg)[:, None])
    return hf


def _mm_bias_kernel(x_ref, w_ref, b_ref, o_ref, *, act):
    y = jnp.dot(x_ref[...], w_ref[...],
                preferred_element_type=jnp.float32) + b_ref[...]
    if act == "relu":
        y = jnp.maximum(y, 0.0)
    o_ref[...] = y


def _mm_bias(x, w, b, act="none", bm=1024):
    M, K = x.shape
    Nc = w.shape[1]
    if M <= bm:
        bm = M
    grid = (M // bm,)
    return pl.pallas_call(
        functools.partial(_mm_bias_kernel, act=act),
        grid=grid,
        in_specs=[
            pl.BlockSpec((bm, K), lambda i: (i, 0)),
            pl.BlockSpec((K, Nc), lambda i: (0, 0)),
            pl.BlockSpec((1, Nc), lambda i: (0, 0)),
        ],
        out_specs=pl.BlockSpec((bm, Nc), lambda i: (i, 0)),
        out_shape=jax.ShapeDtypeStruct((M, Nc), jnp.float32),
    )(x, w, b.reshape(1, Nc))


def _gru_math(gi, gh, h, H):
    r = jax.nn.sigmoid(gi[:, :H] + gh[:, :H])
    z = jax.nn.sigmoid(gi[:, H:2 * H] + gh[:, H:2 * H])
    n = jnp.tanh(gi[:, 2 * H:] + r * gh[:, 2 * H:])
    return (1.0 - z) * n + z * h


def _seq_scan_kernel(gif_ref, gib_ref, whf_ref, whb_ref, bhf_ref, bhb_ref,
                     o_ref, hf, hb, sumf, sumb, maxf, maxb):
    t = pl.program_id(0)

    @pl.when(t == 0)
    def _():
        hf[...] = jnp.zeros_like(hf)
        hb[...] = jnp.zeros_like(hb)
        sumf[...] = jnp.zeros_like(sumf)
        sumb[...] = jnp.zeros_like(sumb)
        maxf[...] = jnp.full_like(maxf, -jnp.inf)
        maxb[...] = jnp.full_like(maxb, -jnp.inf)

    ghf = jnp.dot(hf[...], whf_ref[...],
                  preferred_element_type=jnp.float32) + bhf_ref[...]
    hfn = _gru_math(gif_ref[0], ghf, hf[...], HS)
    hf[...] = hfn
    sumf[...] += hfn
    maxf[...] = jnp.maximum(maxf[...], hfn)

    ghb = jnp.dot(hb[...], whb_ref[...],
                  preferred_element_type=jnp.float32) + bhb_ref[...]
    hbn = _gru_math(gib_ref[0], ghb, hb[...], HS)
    hb[...] = hbn
    sumb[...] += hbn
    maxb[...] = jnp.maximum(maxb[...], hbn)

    @pl.when(t == T - 1)
    def _():
        o_ref[0] = sumf[...]
        o_ref[1] = sumb[...]
        o_ref[2] = maxf[...]
        o_ref[3] = maxb[...]


def _seq_branch(seq, p):
    x2d = seq.reshape(B * T, D)
    wf = jnp.concatenate([p['sWih_f'].T, p['sWih_b'].T], axis=1)
    bf = jnp.concatenate([p['sbih_f'], p['sbih_b']])
    gi = _mm_bias(x2d, wf, bf)
    gi = gi.reshape(B, T, 2, 3 * HS).transpose(2, 1, 0, 3)
    gif, gib = gi[0], gi[1]
    out = pl.pallas_call(
        _seq_scan_kernel,
        grid=(T,),
        in_specs=[
            pl.BlockSpec((1, B, 3 * HS), lambda t: (t, 0, 0)),
            pl.BlockSpec((1, B, 3 * HS), lambda t: (T - 1 - t, 0, 0)),
            pl.BlockSpec((HS, 3 * HS), lambda t: (0, 0)),
            pl.BlockSpec((HS, 3 * HS), lambda t: (0, 0)),
            pl.BlockSpec((1, 3 * HS), lambda t: (0, 0)),
            pl.BlockSpec((1, 3 * HS), lambda t: (0, 0)),
        ],
        out_specs=pl.BlockSpec((4, B, HS), lambda t: (0, 0, 0)),
        out_shape=jax.ShapeDtypeStruct((4, B, HS), jnp.float32),
        scratch_shapes=[pltpu.VMEM((B, HS), jnp.float32)] * 6,
    )(gif, gib, p['sWhh_f'].T, p['sWhh_b'].T,
      p['sbhh_f'].reshape(1, 3 * HS), p['sbhh_b'].reshape(1, 3 * HS))
    seq1 = jnp.concatenate([out[0], out[1]], axis=1) / T
    seq2 = jnp.concatenate([out[2], out[3]], axis=1)
    return seq1, seq2


def _graph_scan_kernel(gif_ref, gib_ref, whf_ref, whb_ref, bhf_ref, bhb_ref,
                       yf_ref, yb_ref, hf, hb):
    t = pl.program_id(0)

    @pl.when(t == 0)
    def _():
        hf[...] = jnp.zeros_like(hf)
        hb[...] = jnp.zeros_like(hb)

    ghf = jnp.dot(hf[...], whf_ref[...],
                  preferred_element_type=jnp.float32) + bhf_ref[...]
    hfn = _gru_math(gif_ref[0], ghf, hf[...], HG)
    hf[...] = hfn
    yf_ref[0] = hfn

    ghb = jnp.dot(hb[...], whb_ref[...],
                  preferred_element_type=jnp.float32) + bhb_ref[...]
    hbn = _gru_math(gib_ref[0], ghb, hb[...], HG)
    hb[...] = hbn
    yb_ref[0] = hbn


def _graph_branch(features, p):
    wf = jnp.concatenate([p['gWih_f'].T, p['gWih_b'].T], axis=1)
    bf = jnp.concatenate([p['gbih_f'], p['gbih_b']])
    gi = _mm_bias(features, wf, bf)
    gi = gi.reshape(B, NPG, 2, 3 * HG).transpose(2, 1, 0, 3)
    gif, gib = gi[0], gi[1]
    yf, yb = pl.pallas_call(
        _graph_scan_kernel,
        grid=(NPG,),
        in_specs=[
            pl.BlockSpec((1, B, 3 * HG), lambda t: (t, 0, 0)),
            pl.BlockSpec((1, B, 3 * HG), lambda t: (NPG - 1 - t, 0, 0)),
            pl.BlockSpec((HG, 3 * HG), lambda t: (0, 0)),
            pl.BlockSpec((HG, 3 * HG), lambda t: (0, 0)),
            pl.BlockSpec((1, 3 * HG), lambda t: (0, 0)),
            pl.BlockSpec((1, 3 * HG), lambda t: (0, 0)),
        ],
        out_specs=[
            pl.BlockSpec((1, B, HG), lambda t: (t, 0, 0)),
            pl.BlockSpec((1, B, HG), lambda t: (NPG - 1 - t, 0, 0)),
        ],
        out_shape=[
            jax.ShapeDtypeStruct((NPG, B, HG), jnp.float32),
            jax.ShapeDtypeStruct((NPG, B, HG), jnp.float32),
        ],
        scratch_shapes=[pltpu.VMEM((B, HG), jnp.float32)] * 2,
    )(gif, gib, p['gWhh_f'].T, p['gWhh_b'].T,
      p['gbhh_f'].reshape(1, 3 * HG), p['gbhh_b'].reshape(1, 3 * HG))
    st = jnp.concatenate([yf, yb], axis=2).transpose(1, 0, 2)
    return st.reshape(N, 2 * HG)


def _mlp_readout(x, W0, b0, W1, b1, W2, b2):
    h = _mm_bias(x, W0.T, b0, act="relu")
    h = _mm_bias(h, W1.T, b1, act="relu")
    return _mm_bias(h, W2.T, b2)


def kernel(features, edge_index, seq, params):
    p = params
    seq1, seq2 = _seq_branch(seq, p)
    h = _graph_branch(features, p)
    h = _appnp_sc(h, edge_index)
    stg = h.reshape(B, NPG, 2 * HG)
    st1 = jnp.max(stg, axis=1)
    st2 = jnp.mean(stg, axis=1)
    outputs = _mlp_readout(st1 + st2, p['mW0'], p['mb0'], p['mW1'], p['mb1'],
                           p['mW2'], p['mb2'])
    outputs1 = _mlp_readout(seq1 + seq2, p['nW0'], p['nb0'], p['nW1'], p['nb1'],
                            p['nW2'], p['nb2'])
    out = outputs1 + outputs
    return (out, out, out)
